# baseline (speedup 1.0000x reference)
import functools
import jax
import jax.numpy as jnp
from jax import lax
from jax.experimental import pallas as pl
from jax.experimental.pallas import tpu as pltpu
from jax.experimental.pallas import tpu_sc as plsc

NUM_GENES = 50000
NUM_EDGES = 10000
NNZ = 500000
EMBED = 128
LATENT = 128
NUM_TREAT = 100
NUM_PATH = 50
P_PAD = 64
B = 1024
M = 200

TB = 32

NC = 2
NS = 16
NW = NC * NS
PER_TILE = NNZ // NW
NCH = 124
TILE_PAD = NCH * 128 - PER_TILE
N_DUMP = 16
EACC = 10240


def _hop1_body(x_hbm, rows_hbm, cols_hbm, out_hbm,
               rows_v, cols_v, buf0, hx_sh, sem):
    cid = lax.axis_index("c")
    tid = lax.axis_index("s")
    wid = tid * NC + cid

    pltpu.sync_copy(rows_hbm.at[wid], rows_v)
    pltpu.sync_copy(cols_hbm.at[wid], cols_v)

    zeros = jnp.zeros((16,), jnp.float32)

    def zrow(r, _):
        for k in range(8):
            buf0[r, pl.ds(k * 16, 16)] = zeros
        return 0
    lax.fori_loop(0, 128, zrow, 0)

    base = tid * (EACC // NS)
    for s in range(5):
        pltpu.sync_copy(buf0, hx_sh.at[pl.ds(base + s * 128, 128)])
    plsc.subcore_barrier()

    def chunk(j, _):
        pltpu.async_copy(x_hbm.at[rows_v.at[j]], buf0, sem).wait()
        pltpu.sync_copy(buf0, hx_sh.at[cols_v.at[j]], add=True)
        return 0
    lax.fori_loop(0, NCH, chunk, 0)

    plsc.subcore_barrier()
    wbase = tid * (EACC // NS)
    pltpu.sync_copy(hx_sh.at[pl.ds(wbase, EACC // NS)],
                    out_hbm.at[cid, pl.ds(wbase, EACC // NS)])


def _hop1(x, rows_t, cols_t):
    mesh = plsc.VectorSubcoreMesh(core_axis_name="c", subcore_axis_name="s")
    return pl.kernel(
        _hop1_body,
        out_type=jax.ShapeDtypeStruct((NC, EACC, EMBED), jnp.float32),
        mesh=mesh,
        scratch_types=[
            pltpu.VMEM((NCH, 128), jnp.int32),
            pltpu.VMEM((NCH, 128), jnp.int32),
            pltpu.VMEM((128, EMBED), jnp.float32),
            pltpu.VMEM_SHARED((EACC, EMBED), jnp.float32),
            pltpu.SemaphoreType.DMA,
        ],
    )(x, rows_t, cols_t)


GDUMP = 50176
GACC2 = 50304
DSL = 16
NPASS = EMBED // DSL


def _hop2_body(hxt, g_hbm, s_hbm, out_hbm, g_v, s_v, buf0, zbuf, acc_sh, sem):
    cid = lax.axis_index("c")
    tid = lax.axis_index("s")
    wid = tid * NC + cid

    pltpu.sync_copy(s_hbm.at[wid], s_v)

    zeros = jnp.zeros((16,), jnp.float32)

    def zrow(r, _):
        for k in range(DSL // 16):
            zbuf[r, pl.ds(k * 16, 16)] = zeros
        return 0
    lax.fori_loop(0, 128, zrow, 0)

    zslice = GACC2 // NS
    for p in range(NPASS):
        pltpu.sync_copy(g_hbm.at[p, wid], g_v)
        zbase = tid * zslice
        for sstep in range(zslice // 128):
            pltpu.sync_copy(zbuf, acc_sh.at[pl.ds(zbase + sstep * 128, 128)])
        rem = zslice % 128
        if rem:
            pltpu.sync_copy(zbuf.at[pl.ds(0, rem)],
                            acc_sh.at[pl.ds(zbase + (zslice // 128) * 128, rem)])
        plsc.subcore_barrier()

        def trip(j, _):
            pltpu.async_copy(hxt.at[g_v.at[j]], buf0, sem).wait()
            pltpu.sync_copy(buf0, acc_sh.at[s_v.at[j]], add=True)
            return 0
        lax.fori_loop(0, NCH, trip, 0)

        plsc.subcore_barrier()
        pltpu.sync_copy(acc_sh.at[pl.ds(tid * zslice, zslice)],
                        out_hbm.at[cid, p, pl.ds(tid * zslice, zslice)])
        plsc.subcore_barrier()


def _hop2(hxt, g_all, s3):
    mesh = plsc.VectorSubcoreMesh(core_axis_name="c", subcore_axis_name="s")
    return pl.kernel(
        _hop2_body,
        out_type=jax.ShapeDtypeStruct((NC, NPASS, GACC2, DSL), jnp.float32),
        mesh=mesh,
        compiler_params=pltpu.CompilerParams(use_tc_tiling_on_sc=False),
        scratch_types=[
            pltpu.VMEM((NCH, 128), jnp.int32),
            pltpu.VMEM((NCH, 128), jnp.int32),
            pltpu.VMEM((128, DSL), jnp.float32),
            pltpu.VMEM((128, DSL), jnp.float32),
            pltpu.VMEM_SHARED((GACC2, DSL), jnp.float32),
            pltpu.SemaphoreType.DMA,
        ],
    )(hxt, g_all, s3)


def _pad_pairs(gather_idx, scatter_idx, gather_mod, dump_base):
    pad_g = (jnp.arange(TILE_PAD, dtype=jnp.int32) * 97) % gather_mod
    pad_g = jnp.broadcast_to(pad_g[None, :], (NW, TILE_PAD))
    pad_s = dump_base + (jnp.arange(TILE_PAD, dtype=jnp.int32) % N_DUMP)
    pad_s = jnp.broadcast_to(pad_s[None, :], (NW, TILE_PAD))
    g3 = jnp.concatenate(
        [gather_idx.reshape(NW, PER_TILE), pad_g], axis=1).reshape(NW, NCH, 128)
    s3 = jnp.concatenate(
        [scatter_idx.reshape(NW, PER_TILE), pad_s], axis=1).reshape(NW, NCH, 128)
    return g3, s3


def _dense_body(xg_ref, pmask_ref, ctx_ref,
                w1a_ref, w1b_ref, b1_ref, w2_ref, b2_ref,
                lw_ref, lb_ref, rw_ref, rb_ref,
                risk_ref, z_ref):
    ctx = ctx_ref[...]
    ctx_h = jnp.dot(ctx, w1b_ref[...], preferred_element_type=jnp.float32)

    def one_batch(b):
        xg = xg_ref[b]
        pm = pmask_ref[b]
        pgs = lax.dot_general(pm, xg, (((0,), (0,)), ((), ())),
                              preferred_element_type=jnp.float32)
        counts = jnp.clip(jnp.sum(pm, axis=0), 1.0, None)
        reps = pgs / counts[:, None]
        h = jnp.tanh(jnp.dot(reps, w1a_ref[...],
                             preferred_element_type=jnp.float32)
                     + ctx_h[b][None, :] + b1_ref[...])
        scores = jnp.dot(h, w2_ref[...],
                         preferred_element_type=jnp.float32)[:, 0] + b2_ref[0, 0]
        pid = lax.broadcasted_iota(jnp.int32, (P_PAD,), 0)
        scores = jnp.where(pid < NUM_PATH, scores, -jnp.inf)
        scores = scores - jnp.max(scores)
        e = jnp.exp(scores)
        w = e / jnp.sum(e)
        z = jnp.dot(w[None, :], reps,
                    preferred_element_type=jnp.float32)
        z_ref[b, :] = z[0]

    for b in range(TB):
        one_batch(b)
    zlat = (jnp.dot(z_ref[...], lw_ref[...], preferred_element_type=jnp.float32)
            + lb_ref[...])
    z_ref[...] = zlat
    risk_ref[...] = (jnp.dot(zlat, rw_ref[...],
                             preferred_element_type=jnp.float32)
                     + rb_ref[0, 0])


def _dense_stage(xg, pmask, ctx, path_w1, path_b1, path_w2, path_b2,
                 latent_w, latent_b, risk_w, risk_b):
    w1a = path_w1[:EMBED]
    w1b = path_w1[EMBED:]
    grid = (B // TB,)
    flt = jnp.float32
    risk, z = pl.pallas_call(
        _dense_body,
        grid=grid,
        in_specs=[
            pl.BlockSpec((TB, M, EMBED), lambda i: (i, 0, 0)),
            pl.BlockSpec((TB, M, P_PAD), lambda i: (i, 0, 0)),
            pl.BlockSpec((TB, EMBED), lambda i: (i, 0)),
            pl.BlockSpec((EMBED, EMBED), lambda i: (0, 0)),
            pl.BlockSpec((EMBED, EMBED), lambda i: (0, 0)),
            pl.BlockSpec((EMBED,), lambda i: (0,)),
            pl.BlockSpec((EMBED, 1), lambda i: (0, 0)),
            pl.BlockSpec((1, 1), lambda i: (0, 0)),
            pl.BlockSpec((EMBED, LATENT), lambda i: (0, 0)),
            pl.BlockSpec((LATENT,), lambda i: (0,)),
            pl.BlockSpec((LATENT, 1), lambda i: (0, 0)),
            pl.BlockSpec((1, 1), lambda i: (0, 0)),
        ],
        out_specs=[
            pl.BlockSpec((TB, 1), lambda i: (i, 0)),
            pl.BlockSpec((TB, LATENT), lambda i: (i, 0)),
        ],
        out_shape=[
            jax.ShapeDtypeStruct((B, 1), flt),
            jax.ShapeDtypeStruct((B, LATENT), flt),
        ],
    )(xg, pmask, ctx, w1a, w1b, path_b1, path_w2,
      path_b2.reshape(1, 1), latent_w, latent_b, risk_w, risk_b.reshape(1, 1))
    return risk[:, 0], z


def kernel(gene_ids, context_ids, gene_embed, treat_embed, h_rows, h_cols,
           h_vals, gene_pathway, path_w1, path_b1, path_w2, path_b2,
           latent_w, latent_b, risk_w, risk_b):
    Dv = jax.ops.segment_sum(h_vals, h_rows, num_segments=NUM_GENES)
    De = jax.ops.segment_sum(h_vals, h_cols, num_segments=NUM_EDGES)
    Dv_inv_sqrt = jnp.power(Dv + 1e-06, -0.5)[:, None]
    De_inv = jnp.power(De + 1e-06, -1.0)[:, None]
    X = gene_embed * Dv_inv_sqrt
    g1, s1 = _pad_pairs(h_rows, h_cols, NUM_GENES, NUM_EDGES)
    hx_part = _hop1(X, g1, s1)
    HX = (hx_part[0, :NUM_EDGES] + hx_part[1, :NUM_EDGES]) * De_inv
    g2, s2 = _pad_pairs(h_cols, h_rows, NUM_EDGES, GDUMP)
    hxt = HX.reshape(NUM_EDGES, NPASS, DSL).transpose(1, 0, 2).reshape(
        NPASS * NUM_EDGES, DSL)
    g_all = g2[None] + (jnp.arange(NPASS, dtype=jnp.int32)
                        * NUM_EDGES)[:, None, None, None]
    xp_part = _hop2(hxt, g_all, s2)
    xp = (xp_part[0] + xp_part[1]).transpose(1, 0, 2).reshape(GACC2, EMBED)
    X_prop = xp[:NUM_GENES] * Dv_inv_sqrt

    xg = X_prop[gene_ids]
    ctx = treat_embed[context_ids]
    pmask = gene_pathway[gene_ids]
    pmask = jnp.pad(pmask, ((0, 0), (0, 0), (0, P_PAD - NUM_PATH)))

    return _dense_stage(xg, pmask, ctx, path_w1, path_b1, path_w2, path_b2,
                        latent_w, latent_b, risk_w, risk_b)


# speedup vs baseline: 1.2896x; 1.2896x over previous
import functools
import jax
import jax.numpy as jnp
from jax import lax
from jax.experimental import pallas as pl
from jax.experimental.pallas import tpu as pltpu
from jax.experimental.pallas import tpu_sc as plsc

NUM_GENES = 50000
NUM_EDGES = 10000
NNZ = 500000
EMBED = 128
LATENT = 128
NUM_TREAT = 100
NUM_PATH = 50
P_PAD = 64
B = 1024
M = 200

TB = 32

NC = 2
NS = 16
NW = NC * NS
PER_TILE = NNZ // NW
NCH = 124
TILE_PAD = NCH * 128 - PER_TILE
N_DUMP = 16
EACC = 10240


def _hop1_body(x_hbm, rows_hbm, cols_hbm, out_hbm,
               rows_v, cols_v, buf0, hx_sh, sem):
    cid = lax.axis_index("c")
    tid = lax.axis_index("s")
    wid = tid * NC + cid

    pltpu.sync_copy(rows_hbm.at[wid], rows_v)
    pltpu.sync_copy(cols_hbm.at[wid], cols_v)

    zeros = jnp.zeros((16,), jnp.float32)

    def zrow(r, _):
        for k in range(8):
            buf0[r, pl.ds(k * 16, 16)] = zeros
        return 0
    lax.fori_loop(0, 128, zrow, 0)

    base = tid * (EACC // NS)
    for s in range(5):
        pltpu.sync_copy(buf0, hx_sh.at[pl.ds(base + s * 128, 128)])
    plsc.subcore_barrier()

    def chunk(j, _):
        pltpu.async_copy(x_hbm.at[rows_v.at[j]], buf0, sem).wait()
        pltpu.sync_copy(buf0, hx_sh.at[cols_v.at[j]], add=True)
        return 0
    lax.fori_loop(0, NCH, chunk, 0)

    plsc.subcore_barrier()
    wbase = tid * (EACC // NS)
    pltpu.sync_copy(hx_sh.at[pl.ds(wbase, EACC // NS)],
                    out_hbm.at[cid, pl.ds(wbase, EACC // NS)])


def _hop1(x, rows_t, cols_t):
    mesh = plsc.VectorSubcoreMesh(core_axis_name="c", subcore_axis_name="s")
    return pl.kernel(
        _hop1_body,
        out_type=jax.ShapeDtypeStruct((NC, EACC, EMBED), jnp.float32),
        mesh=mesh,
        scratch_types=[
            pltpu.VMEM((NCH, 128), jnp.int32),
            pltpu.VMEM((NCH, 128), jnp.int32),
            pltpu.VMEM((128, EMBED), jnp.float32),
            pltpu.VMEM_SHARED((EACC, EMBED), jnp.float32),
            pltpu.SemaphoreType.DMA,
        ],
    )(x, rows_t, cols_t)


GDUMP = 50176
GACC2 = 50304
DSL = 16
NPASS = EMBED // DSL


def _hop2_body(hxt, g_hbm, s_hbm, out_hbm, g_v, s_v, buf0, zbuf, acc_sh, sem):
    cid = lax.axis_index("c")
    tid = lax.axis_index("s")
    wid = tid * NC + cid

    pltpu.sync_copy(s_hbm.at[wid], s_v)

    zeros = jnp.zeros((16,), jnp.float32)

    def zrow(r, _):
        for k in range(DSL // 16):
            zbuf[r, pl.ds(k * 16, 16)] = zeros
        return 0
    lax.fori_loop(0, 128, zrow, 0)

    zslice = GACC2 // NS
    for p in range(NPASS):
        pltpu.sync_copy(g_hbm.at[p, wid], g_v)
        zbase = tid * zslice
        for sstep in range(zslice // 128):
            pltpu.sync_copy(zbuf, acc_sh.at[pl.ds(zbase + sstep * 128, 128)])
        rem = zslice % 128
        if rem:
            pltpu.sync_copy(zbuf.at[pl.ds(0, rem)],
                            acc_sh.at[pl.ds(zbase + (zslice // 128) * 128, rem)])
        plsc.subcore_barrier()

        def trip(j, _):
            pltpu.async_copy(hxt.at[g_v.at[j]], buf0, sem).wait()
            pltpu.sync_copy(buf0, acc_sh.at[s_v.at[j]], add=True)
            return 0
        lax.fori_loop(0, NCH, trip, 0)

        plsc.subcore_barrier()
        pltpu.sync_copy(acc_sh.at[pl.ds(tid * zslice, zslice)],
                        out_hbm.at[cid, p, pl.ds(tid * zslice, zslice)])
        plsc.subcore_barrier()


def _hop2(hxt, g_all, s3):
    mesh = plsc.VectorSubcoreMesh(core_axis_name="c", subcore_axis_name="s")
    return pl.kernel(
        _hop2_body,
        out_type=jax.ShapeDtypeStruct((NC, NPASS, GACC2, DSL), jnp.float32),
        mesh=mesh,
        compiler_params=pltpu.CompilerParams(use_tc_tiling_on_sc=False),
        scratch_types=[
            pltpu.VMEM((NCH, 128), jnp.int32),
            pltpu.VMEM((NCH, 128), jnp.int32),
            pltpu.VMEM((128, DSL), jnp.float32),
            pltpu.VMEM((128, DSL), jnp.float32),
            pltpu.VMEM_SHARED((GACC2, DSL), jnp.float32),
            pltpu.SemaphoreType.DMA,
        ],
    )(hxt, g_all, s3)


def _pad_pairs(gather_idx, scatter_idx, gather_mod, dump_base):
    pad_g = (jnp.arange(TILE_PAD, dtype=jnp.int32) * 97) % gather_mod
    pad_g = jnp.broadcast_to(pad_g[None, :], (NW, TILE_PAD))
    pad_s = dump_base + (jnp.arange(TILE_PAD, dtype=jnp.int32) % N_DUMP)
    pad_s = jnp.broadcast_to(pad_s[None, :], (NW, TILE_PAD))
    g3 = jnp.concatenate(
        [gather_idx.reshape(NW, PER_TILE), pad_g], axis=1).reshape(NW, NCH, 128)
    s3 = jnp.concatenate(
        [scatter_idx.reshape(NW, PER_TILE), pad_s], axis=1).reshape(NW, NCH, 128)
    return g3, s3


NB_IDS = B * M
PER_TILE_B = NB_IDS // NW
NCH_B = PER_TILE_B // 128
PER_TILE_C = B // NW


def _bgather_body(xp_hbm, gp_hbm, te_hbm, gid_hbm, cid_hbm,
                  xg_out, pm_out, ctx_out,
                  gid_v, cid_v, bufx, bufp, bufc, semx, semp):
    cid = lax.axis_index("c")
    tid = lax.axis_index("s")
    wid = tid * NC + cid

    pltpu.sync_copy(gid_hbm.at[wid], gid_v)
    pltpu.sync_copy(cid_hbm.at[wid], cid_v)

    pltpu.async_copy(te_hbm.at[cid_v], bufc, semx).wait()
    pltpu.sync_copy(bufc, ctx_out.at[pl.ds(wid * PER_TILE_C, PER_TILE_C)])

    obase = wid * PER_TILE_B

    def chunk(j, _):
        dx = pltpu.async_copy(xp_hbm.at[gid_v.at[j]], bufx, semx)
        dp = pltpu.async_copy(gp_hbm.at[gid_v.at[j]], bufp, semp)
        dx.wait()
        pltpu.sync_copy(bufx, xg_out.at[pl.ds(obase + j * 128, 128)])
        dp.wait()
        pltpu.sync_copy(bufp, pm_out.at[pl.ds(obase + j * 128, 128)])
        return 0
    lax.fori_loop(0, NCH_B, chunk, 0)


def _bgather(x_prop, gp_pad, treat_embed, gid3, cid2):
    mesh = plsc.VectorSubcoreMesh(core_axis_name="c", subcore_axis_name="s")
    return pl.kernel(
        _bgather_body,
        out_type=[
            jax.ShapeDtypeStruct((NB_IDS, EMBED), jnp.float32),
            jax.ShapeDtypeStruct((NB_IDS, P_PAD), jnp.float32),
            jax.ShapeDtypeStruct((B, EMBED), jnp.float32),
        ],
        mesh=mesh,
        compiler_params=pltpu.CompilerParams(use_tc_tiling_on_sc=False),
        scratch_types=[
            pltpu.VMEM((NCH_B, 128), jnp.int32),
            pltpu.VMEM((PER_TILE_C,), jnp.int32),
            pltpu.VMEM((128, EMBED), jnp.float32),
            pltpu.VMEM((128, P_PAD), jnp.float32),
            pltpu.VMEM((PER_TILE_C, EMBED), jnp.float32),
            pltpu.SemaphoreType.DMA,
            pltpu.SemaphoreType.DMA,
        ],
    )(x_prop, gp_pad, treat_embed, gid3, cid2)


def _dense_body(xg_ref, pmask_ref, ctx_ref,
                w1a_ref, w1b_ref, b1_ref, w2_ref, b2_ref,
                lw_ref, lb_ref, rw_ref, rb_ref,
                risk_ref, z_ref):
    ctx = ctx_ref[...]
    ctx_h = jnp.dot(ctx, w1b_ref[...], preferred_element_type=jnp.float32)

    def one_batch(b):
        xg = xg_ref[b]
        pm = pmask_ref[b]
        pgs = lax.dot_general(pm, xg, (((0,), (0,)), ((), ())),
                              preferred_element_type=jnp.float32)
        counts = jnp.clip(jnp.sum(pm, axis=0), 1.0, None)
        reps = pgs / counts[:, None]
        h = jnp.tanh(jnp.dot(reps, w1a_ref[...],
                             preferred_element_type=jnp.float32)
                     + ctx_h[b][None, :] + b1_ref[...])
        scores = jnp.dot(h, w2_ref[...],
                         preferred_element_type=jnp.float32)[:, 0] + b2_ref[0, 0]
        pid = lax.broadcasted_iota(jnp.int32, (P_PAD,), 0)
        scores = jnp.where(pid < NUM_PATH, scores, -jnp.inf)
        scores = scores - jnp.max(scores)
        e = jnp.exp(scores)
        w = e / jnp.sum(e)
        z = jnp.dot(w[None, :], reps,
                    preferred_element_type=jnp.float32)
        z_ref[b, :] = z[0]

    for b in range(TB):
        one_batch(b)
    zlat = (jnp.dot(z_ref[...], lw_ref[...], preferred_element_type=jnp.float32)
            + lb_ref[...])
    z_ref[...] = zlat
    risk_ref[...] = (jnp.dot(zlat, rw_ref[...],
                             preferred_element_type=jnp.float32)
                     + rb_ref[0, 0])


def _dense_stage(xg, pmask, ctx, path_w1, path_b1, path_w2, path_b2,
                 latent_w, latent_b, risk_w, risk_b):
    w1a = path_w1[:EMBED]
    w1b = path_w1[EMBED:]
    grid = (B // TB,)
    flt = jnp.float32
    risk, z = pl.pallas_call(
        _dense_body,
        grid=grid,
        in_specs=[
            pl.BlockSpec((TB, M, EMBED), lambda i: (i, 0, 0)),
            pl.BlockSpec((TB, M, P_PAD), lambda i: (i, 0, 0)),
            pl.BlockSpec((TB, EMBED), lambda i: (i, 0)),
            pl.BlockSpec((EMBED, EMBED), lambda i: (0, 0)),
            pl.BlockSpec((EMBED, EMBED), lambda i: (0, 0)),
            pl.BlockSpec((EMBED,), lambda i: (0,)),
            pl.BlockSpec((EMBED, 1), lambda i: (0, 0)),
            pl.BlockSpec((1, 1), lambda i: (0, 0)),
            pl.BlockSpec((EMBED, LATENT), lambda i: (0, 0)),
            pl.BlockSpec((LATENT,), lambda i: (0,)),
            pl.BlockSpec((LATENT, 1), lambda i: (0, 0)),
            pl.BlockSpec((1, 1), lambda i: (0, 0)),
        ],
        out_specs=[
            pl.BlockSpec((TB, 1), lambda i: (i, 0)),
            pl.BlockSpec((TB, LATENT), lambda i: (i, 0)),
        ],
        out_shape=[
            jax.ShapeDtypeStruct((B, 1), flt),
            jax.ShapeDtypeStruct((B, LATENT), flt),
        ],
    )(xg, pmask, ctx, w1a, w1b, path_b1, path_w2,
      path_b2.reshape(1, 1), latent_w, latent_b, risk_w, risk_b.reshape(1, 1))
    return risk[:, 0], z


def kernel(gene_ids, context_ids, gene_embed, treat_embed, h_rows, h_cols,
           h_vals, gene_pathway, path_w1, path_b1, path_w2, path_b2,
           latent_w, latent_b, risk_w, risk_b):
    Dv = jax.ops.segment_sum(h_vals, h_rows, num_segments=NUM_GENES)
    De = jax.ops.segment_sum(h_vals, h_cols, num_segments=NUM_EDGES)
    Dv_inv_sqrt = jnp.power(Dv + 1e-06, -0.5)[:, None]
    De_inv = jnp.power(De + 1e-06, -1.0)[:, None]
    X = gene_embed * Dv_inv_sqrt
    g1, s1 = _pad_pairs(h_rows, h_cols, NUM_GENES, NUM_EDGES)
    hx_part = _hop1(X, g1, s1)
    HX = (hx_part[0, :NUM_EDGES] + hx_part[1, :NUM_EDGES]) * De_inv
    g2, s2 = _pad_pairs(h_cols, h_rows, NUM_EDGES, GDUMP)
    hxt = HX.reshape(NUM_EDGES, NPASS, DSL).transpose(1, 0, 2).reshape(
        NPASS * NUM_EDGES, DSL)
    g_all = g2[None] + (jnp.arange(NPASS, dtype=jnp.int32)
                        * NUM_EDGES)[:, None, None, None]
    xp_part = _hop2(hxt, g_all, s2)
    xp = (xp_part[0] + xp_part[1]).transpose(1, 0, 2).reshape(GACC2, EMBED)
    X_prop = xp[:NUM_GENES] * Dv_inv_sqrt

    gp_pad = jnp.pad(gene_pathway, ((0, 0), (0, P_PAD - NUM_PATH)))
    gid3 = gene_ids.reshape(NW, NCH_B, 128)
    cid2 = context_ids.reshape(NW, PER_TILE_C)
    xg_flat, pm_flat, ctx = _bgather(X_prop, gp_pad, treat_embed, gid3, cid2)
    xg = xg_flat.reshape(B, M, EMBED)
    pmask = pm_flat.reshape(B, M, P_PAD)

    return _dense_stage(xg, pmask, ctx, path_w1, path_b1, path_w2, path_b2,
                        latent_w, latent_b, risk_w, risk_b)


# speedup vs baseline: 1.4924x; 1.1572x over previous
import functools
import jax
import jax.numpy as jnp
from jax import lax
from jax.experimental import pallas as pl
from jax.experimental.pallas import tpu as pltpu
from jax.experimental.pallas import tpu_sc as plsc

NUM_GENES = 50000
NUM_EDGES = 10000
NNZ = 500000
EMBED = 128
LATENT = 128
NUM_TREAT = 100
NUM_PATH = 50
P_PAD = 64
B = 1024
M = 200

TB = 32

NC = 2
NS = 16
NW = NC * NS
PER_TILE = NNZ // NW
NCH = 124
TILE_PAD = NCH * 128 - PER_TILE
N_DUMP = 16
EACC = 10240


def _hop1_body(x_hbm, rows_hbm, cols_hbm, out_hbm,
               rows_v, cols_v, buf0, hx_sh, sem):
    cid = lax.axis_index("c")
    tid = lax.axis_index("s")
    wid = tid * NC + cid

    pltpu.sync_copy(rows_hbm.at[wid], rows_v)
    pltpu.sync_copy(cols_hbm.at[wid], cols_v)

    zeros = jnp.zeros((16,), jnp.float32)

    def zrow(r, _):
        for k in range(8):
            buf0[r, pl.ds(k * 16, 16)] = zeros
        return 0
    lax.fori_loop(0, 128, zrow, 0)

    base = tid * (EACC // NS)
    for s in range(5):
        pltpu.sync_copy(buf0, hx_sh.at[pl.ds(base + s * 128, 128)])
    plsc.subcore_barrier()

    def chunk(j, _):
        pltpu.async_copy(x_hbm.at[rows_v.at[j]], buf0, sem).wait()
        pltpu.sync_copy(buf0, hx_sh.at[cols_v.at[j]], add=True)
        return 0
    lax.fori_loop(0, NCH, chunk, 0)

    plsc.subcore_barrier()
    wbase = tid * (EACC // NS)
    pltpu.sync_copy(hx_sh.at[pl.ds(wbase, EACC // NS)],
                    out_hbm.at[cid, pl.ds(wbase, EACC // NS)])


def _hop1(x, rows_t, cols_t):
    mesh = plsc.VectorSubcoreMesh(core_axis_name="c", subcore_axis_name="s")
    return pl.kernel(
        _hop1_body,
        out_type=jax.ShapeDtypeStruct((NC, EACC, EMBED), jnp.float32),
        mesh=mesh,
        scratch_types=[
            pltpu.VMEM((NCH, 128), jnp.int32),
            pltpu.VMEM((NCH, 128), jnp.int32),
            pltpu.VMEM((128, EMBED), jnp.float32),
            pltpu.VMEM_SHARED((EACC, EMBED), jnp.float32),
            pltpu.SemaphoreType.DMA,
        ],
    )(x, rows_t, cols_t)


GDUMP = 50176
GACC2 = 50304
DSL = 16
NPASS = EMBED // DSL


def _hop2_body(hxt, g_hbm, s_hbm, out_hbm,
               g_v, s_v, bufa, bufb, zbuf, acc_sh, sema, semb):
    cid = lax.axis_index("c")
    tid = lax.axis_index("s")
    wid = tid * NC + cid

    pltpu.sync_copy(g_hbm.at[wid], g_v)
    pltpu.sync_copy(s_hbm.at[wid], s_v)

    zeros = jnp.zeros((16,), jnp.float32)

    def zrow(r, _):
        for k in range(DSL // 16):
            zbuf[r, pl.ds(k * 16, 16)] = zeros
        return 0
    lax.fori_loop(0, 128, zrow, 0)

    zslice = GACC2 // NS
    for p in range(NPASS):
        if p > 0:
            def badd(j, _):
                for k in range(8):
                    g_v[j, pl.ds(k * 16, 16)] = (
                        g_v[j, pl.ds(k * 16, 16)] + NUM_EDGES)
                return 0
            lax.fori_loop(0, NCH, badd, 0)
        zbase = tid * zslice
        for sstep in range(zslice // 128):
            pltpu.sync_copy(zbuf, acc_sh.at[pl.ds(zbase + sstep * 128, 128)])
        rem = zslice % 128
        if rem:
            pltpu.sync_copy(zbuf.at[pl.ds(0, rem)],
                            acc_sh.at[pl.ds(zbase + (zslice // 128) * 128, rem)])
        plsc.subcore_barrier()

        pltpu.async_copy(hxt.at[g_v.at[0]], bufa, sema)

        def trip2(i, _):
            j0 = 2 * i
            j1 = 2 * i + 1
            d1 = pltpu.async_copy(hxt.at[g_v.at[j1]], bufb, semb)
            pltpu.make_async_copy(hxt.at[g_v.at[j0]], bufa, sema).wait()
            pltpu.sync_copy(bufa, acc_sh.at[s_v.at[j0]], add=True)

            @pl.when(j1 + 1 < NCH)
            def _():
                pltpu.async_copy(hxt.at[g_v.at[j1 + 1]], bufa, sema)
            d1.wait()
            pltpu.sync_copy(bufb, acc_sh.at[s_v.at[j1]], add=True)
            return 0
        lax.fori_loop(0, NCH // 2, trip2, 0)

        plsc.subcore_barrier()
        pltpu.sync_copy(acc_sh.at[pl.ds(tid * zslice, zslice)],
                        out_hbm.at[cid, p, pl.ds(tid * zslice, zslice)])
        plsc.subcore_barrier()


def _hop2(hxt, g3, s3):
    mesh = plsc.VectorSubcoreMesh(core_axis_name="c", subcore_axis_name="s")
    return pl.kernel(
        _hop2_body,
        out_type=jax.ShapeDtypeStruct((NC, NPASS, GACC2, DSL), jnp.float32),
        mesh=mesh,
        compiler_params=pltpu.CompilerParams(use_tc_tiling_on_sc=False),
        scratch_types=[
            pltpu.VMEM((NCH, 128), jnp.int32),
            pltpu.VMEM((NCH, 128), jnp.int32),
            pltpu.VMEM((128, DSL), jnp.float32),
            pltpu.VMEM((128, DSL), jnp.float32),
            pltpu.VMEM((128, DSL), jnp.float32),
            pltpu.VMEM_SHARED((GACC2, DSL), jnp.float32),
            pltpu.SemaphoreType.DMA,
            pltpu.SemaphoreType.DMA,
        ],
    )(hxt, g3, s3)


def _pad_pairs(gather_idx, scatter_idx, gather_mod, dump_base):
    pad_g = (jnp.arange(TILE_PAD, dtype=jnp.int32) * 97) % gather_mod
    pad_g = jnp.broadcast_to(pad_g[None, :], (NW, TILE_PAD))
    pad_s = dump_base + (jnp.arange(TILE_PAD, dtype=jnp.int32) % N_DUMP)
    pad_s = jnp.broadcast_to(pad_s[None, :], (NW, TILE_PAD))
    g3 = jnp.concatenate(
        [gather_idx.reshape(NW, PER_TILE), pad_g], axis=1).reshape(NW, NCH, 128)
    s3 = jnp.concatenate(
        [scatter_idx.reshape(NW, PER_TILE), pad_s], axis=1).reshape(NW, NCH, 128)
    return g3, s3


NB_IDS = B * M
PER_TILE_B = NB_IDS // NW
NCH_B = PER_TILE_B // 128
PER_TILE_C = B // NW


def _bgather_body(xp_hbm, gp_hbm, te_hbm, gid_hbm, cid_hbm,
                  xg_out, pm_out, ctx_out,
                  gid_v, cid_v, bufx, bufp, bufc, semx, semp):
    cid = lax.axis_index("c")
    tid = lax.axis_index("s")
    wid = tid * NC + cid

    pltpu.sync_copy(gid_hbm.at[wid], gid_v)
    pltpu.sync_copy(cid_hbm.at[wid], cid_v)

    pltpu.async_copy(te_hbm.at[cid_v], bufc, semx).wait()
    pltpu.sync_copy(bufc, ctx_out.at[pl.ds(wid * PER_TILE_C, PER_TILE_C)])

    obase = wid * PER_TILE_B

    def chunk(j, _):
        dx = pltpu.async_copy(xp_hbm.at[gid_v.at[j]], bufx, semx)
        dp = pltpu.async_copy(gp_hbm.at[gid_v.at[j]], bufp, semp)
        dx.wait()
        pltpu.sync_copy(bufx, xg_out.at[pl.ds(obase + j * 128, 128)])
        dp.wait()
        pltpu.sync_copy(bufp, pm_out.at[pl.ds(obase + j * 128, 128)])
        return 0
    lax.fori_loop(0, NCH_B, chunk, 0)


def _bgather(x_prop, gp_pad, treat_embed, gid3, cid2):
    mesh = plsc.VectorSubcoreMesh(core_axis_name="c", subcore_axis_name="s")
    return pl.kernel(
        _bgather_body,
        out_type=[
            jax.ShapeDtypeStruct((NB_IDS, EMBED), jnp.float32),
            jax.ShapeDtypeStruct((NB_IDS, P_PAD), jnp.float32),
            jax.ShapeDtypeStruct((B, EMBED), jnp.float32),
        ],
        mesh=mesh,
        compiler_params=pltpu.CompilerParams(use_tc_tiling_on_sc=False),
        scratch_types=[
            pltpu.VMEM((NCH_B, 128), jnp.int32),
            pltpu.VMEM((PER_TILE_C,), jnp.int32),
            pltpu.VMEM((128, EMBED), jnp.float32),
            pltpu.VMEM((128, P_PAD), jnp.float32),
            pltpu.VMEM((PER_TILE_C, EMBED), jnp.float32),
            pltpu.SemaphoreType.DMA,
            pltpu.SemaphoreType.DMA,
        ],
    )(x_prop, gp_pad, treat_embed, gid3, cid2)


def _dense_body(xg_ref, pmask_ref, ctx_ref,
                w1a_ref, w1b_ref, b1_ref, w2_ref, b2_ref,
                lw_ref, lb_ref, rw_ref, rb_ref,
                risk_ref, z_ref):
    ctx = ctx_ref[...]
    ctx_h = jnp.dot(ctx, w1b_ref[...], preferred_element_type=jnp.float32)

    xg3 = xg_ref[...]
    pm3 = pmask_ref[...]
    pgs3 = lax.dot_general(pm3, xg3, (((1,), (1,)), ((0,), (0,))),
                           preferred_element_type=jnp.float32)
    counts3 = jnp.clip(jnp.sum(pm3, axis=1), 1.0, None)

    def one_batch(b):
        pgs = pgs3[b]
        reps = pgs / counts3[b][:, None]
        h = jnp.tanh(jnp.dot(reps, w1a_ref[...],
                             preferred_element_type=jnp.float32)
                     + ctx_h[b][None, :] + b1_ref[...])
        scores = jnp.dot(h, w2_ref[...],
                         preferred_element_type=jnp.float32)[:, 0] + b2_ref[0, 0]
        pid = lax.broadcasted_iota(jnp.int32, (P_PAD,), 0)
        scores = jnp.where(pid < NUM_PATH, scores, -jnp.inf)
        scores = scores - jnp.max(scores)
        e = jnp.exp(scores)
        w = e / jnp.sum(e)
        z = jnp.dot(w[None, :], reps,
                    preferred_element_type=jnp.float32)
        z_ref[b, :] = z[0]

    for b in range(TB):
        one_batch(b)
    zlat = (jnp.dot(z_ref[...], lw_ref[...], preferred_element_type=jnp.float32)
            + lb_ref[...])
    z_ref[...] = zlat
    risk_ref[...] = (jnp.dot(zlat, rw_ref[...],
                             preferred_element_type=jnp.float32)
                     + rb_ref[0, 0])


def _dense_stage(xg, pmask, ctx, path_w1, path_b1, path_w2, path_b2,
                 latent_w, latent_b, risk_w, risk_b):
    w1a = path_w1[:EMBED]
    w1b = path_w1[EMBED:]
    grid = (B // TB,)
    flt = jnp.float32
    risk, z = pl.pallas_call(
        _dense_body,
        grid=grid,
        in_specs=[
            pl.BlockSpec((TB, M, EMBED), lambda i: (i, 0, 0)),
            pl.BlockSpec((TB, M, P_PAD), lambda i: (i, 0, 0)),
            pl.BlockSpec((TB, EMBED), lambda i: (i, 0)),
            pl.BlockSpec((EMBED, EMBED), lambda i: (0, 0)),
            pl.BlockSpec((EMBED, EMBED), lambda i: (0, 0)),
            pl.BlockSpec((EMBED,), lambda i: (0,)),
            pl.BlockSpec((EMBED, 1), lambda i: (0, 0)),
            pl.BlockSpec((1, 1), lambda i: (0, 0)),
            pl.BlockSpec((EMBED, LATENT), lambda i: (0, 0)),
            pl.BlockSpec((LATENT,), lambda i: (0,)),
            pl.BlockSpec((LATENT, 1), lambda i: (0, 0)),
            pl.BlockSpec((1, 1), lambda i: (0, 0)),
        ],
        out_specs=[
            pl.BlockSpec((TB, 1), lambda i: (i, 0)),
            pl.BlockSpec((TB, LATENT), lambda i: (i, 0)),
        ],
        out_shape=[
            jax.ShapeDtypeStruct((B, 1), flt),
            jax.ShapeDtypeStruct((B, LATENT), flt),
        ],
    )(xg, pmask, ctx, w1a, w1b, path_b1, path_w2,
      path_b2.reshape(1, 1), latent_w, latent_b, risk_w, risk_b.reshape(1, 1))
    return risk[:, 0], z


def kernel(gene_ids, context_ids, gene_embed, treat_embed, h_rows, h_cols,
           h_vals, gene_pathway, path_w1, path_b1, path_w2, path_b2,
           latent_w, latent_b, risk_w, risk_b):
    Dv = jax.ops.segment_sum(h_vals, h_rows, num_segments=NUM_GENES)
    De = jax.ops.segment_sum(h_vals, h_cols, num_segments=NUM_EDGES)
    Dv_inv_sqrt = jnp.power(Dv + 1e-06, -0.5)[:, None]
    De_inv = jnp.power(De + 1e-06, -1.0)[:, None]
    X = gene_embed * Dv_inv_sqrt
    g1, s1 = _pad_pairs(h_rows, h_cols, NUM_GENES, NUM_EDGES)
    hx_part = _hop1(X, g1, s1)
    HX = (hx_part[0, :NUM_EDGES] + hx_part[1, :NUM_EDGES]) * De_inv
    g2, s2 = _pad_pairs(h_cols, h_rows, NUM_EDGES, GDUMP)
    hxt = HX.reshape(NUM_EDGES, NPASS, DSL).transpose(1, 0, 2).reshape(
        NPASS * NUM_EDGES, DSL)
    xp_part = _hop2(hxt, g2, s2)
    xp = (xp_part[0] + xp_part[1]).transpose(1, 0, 2).reshape(GACC2, EMBED)
    X_prop = xp[:NUM_GENES] * Dv_inv_sqrt

    gp_pad = jnp.pad(gene_pathway, ((0, 0), (0, P_PAD - NUM_PATH)))
    gid3 = gene_ids.reshape(NW, NCH_B, 128)
    cid2 = context_ids.reshape(NW, PER_TILE_C)
    xg_flat, pm_flat, ctx = _bgather(X_prop, gp_pad, treat_embed, gid3, cid2)
    xg = xg_flat.reshape(B, M, EMBED)
    pmask = pm_flat.reshape(B, M, P_PAD)

    return _dense_stage(xg, pmask, ctx, path_w1, path_b1, path_w2, path_b2,
                        latent_w, latent_b, risk_w, risk_b)


# speedup vs baseline: 2.0757x; 1.3908x over previous
import functools
import jax
import jax.numpy as jnp
from jax import lax
from jax.experimental import pallas as pl
from jax.experimental.pallas import tpu as pltpu
from jax.experimental.pallas import tpu_sc as plsc

NUM_GENES = 50000
NUM_EDGES = 10000
NNZ = 500000
EMBED = 128
LATENT = 128
NUM_TREAT = 100
NUM_PATH = 50
P_PAD = 64
B = 1024
M = 200

TB = 32

NC = 2
NS = 16
NW = NC * NS
PER_TILE = NNZ // NW
NCH = 124
TILE_PAD = NCH * 128 - PER_TILE
N_DUMP = 16
EACC = 10240


def _make_prop(nacc, dsl, npass, table_mod):
    zslice = nacc // NS

    def body(tab, g_hbm, s_hbm, out_hbm,
             g_v, s_v, bufa, bufb, zbuf, acc_sh, sema, semb):
        cid = lax.axis_index("c")
        tid = lax.axis_index("s")
        wid = tid * NC + cid

        pltpu.sync_copy(g_hbm.at[wid], g_v)
        pltpu.sync_copy(s_hbm.at[wid], s_v)

        zeros = jnp.zeros((16,), jnp.float32)

        def zrow(r, _):
            for k in range(dsl // 16):
                zbuf[r, pl.ds(k * 16, 16)] = zeros
            return 0
        lax.fori_loop(0, 128, zrow, 0)

        for p in range(npass):
            if p > 0:
                def badd(j, _):
                    for k in range(8):
                        g_v[j, pl.ds(k * 16, 16)] = (
                            g_v[j, pl.ds(k * 16, 16)] + table_mod)
                    return 0
                lax.fori_loop(0, NCH, badd, 0)
            zbase = tid * zslice
            for sstep in range(zslice // 128):
                pltpu.sync_copy(zbuf, acc_sh.at[pl.ds(zbase + sstep * 128, 128)])
            rem = zslice % 128
            if rem:
                pltpu.sync_copy(
                    zbuf.at[pl.ds(0, rem)],
                    acc_sh.at[pl.ds(zbase + (zslice // 128) * 128, rem)])
            plsc.subcore_barrier()

            pltpu.async_copy(tab.at[g_v.at[0]], bufa, sema)

            def trip2(i, _):
                j0 = 2 * i
                j1 = 2 * i + 1
                d1 = pltpu.async_copy(tab.at[g_v.at[j1]], bufb, semb)
                pltpu.make_async_copy(tab.at[g_v.at[j0]], bufa, sema).wait()
                pltpu.sync_copy(bufa, acc_sh.at[s_v.at[j0]], add=True)

                @pl.when(j1 + 1 < NCH)
                def _():
                    pltpu.async_copy(tab.at[g_v.at[j1 + 1]], bufa, sema)
                d1.wait()
                pltpu.sync_copy(bufb, acc_sh.at[s_v.at[j1]], add=True)
                return 0
            lax.fori_loop(0, NCH // 2, trip2, 0)

            plsc.subcore_barrier()
            pltpu.sync_copy(acc_sh.at[pl.ds(tid * zslice, zslice)],
                            out_hbm.at[cid, p, pl.ds(tid * zslice, zslice)])
            plsc.subcore_barrier()

    mesh = plsc.VectorSubcoreMesh(core_axis_name="c", subcore_axis_name="s")

    def run(tab, g3, s3):
        return pl.kernel(
            body,
            out_type=jax.ShapeDtypeStruct((NC, npass, nacc, dsl), jnp.float32),
            mesh=mesh,
            compiler_params=pltpu.CompilerParams(use_tc_tiling_on_sc=False),
            scratch_types=[
                pltpu.VMEM((NCH, 128), jnp.int32),
                pltpu.VMEM((NCH, 128), jnp.int32),
                pltpu.VMEM((128, dsl), jnp.float32),
                pltpu.VMEM((128, dsl), jnp.float32),
                pltpu.VMEM((128, dsl), jnp.float32),
                pltpu.VMEM_SHARED((nacc, dsl), jnp.float32),
                pltpu.SemaphoreType.DMA,
                pltpu.SemaphoreType.DMA,
            ],
        )(tab, g3, s3)
    return run


GDUMP = 50176
GACC2 = 50304
DSL = 16
NPASS = EMBED // DSL
DSL1 = 64
NPASS1 = EMBED // DSL1

_hop1 = _make_prop(EACC, DSL1, NPASS1, NUM_GENES)
_hop2 = _make_prop(GACC2, DSL, NPASS, NUM_EDGES)


DVACC = 51200


def _hist_body(s1_hbm, s2_hbm, de_out, dv_out,
               s1_v, s2_v, ones_v, zbuf, de_sh, dv_sh, sem1, sem2):
    cid = lax.axis_index("c")
    tid = lax.axis_index("s")
    wid = tid * NC + cid

    pltpu.sync_copy(s1_hbm.at[wid], s1_v)
    pltpu.sync_copy(s2_hbm.at[wid], s2_v)

    ones = jnp.full((16,), 1.0, jnp.float32)
    zeros = jnp.zeros((16,), jnp.float32)
    for k in range(8):
        ones_v[pl.ds(k * 16, 16)] = ones

    def zrow(r, _):
        zbuf[pl.ds(r * 16, 16)] = zeros
        return 0
    lax.fori_loop(0, 200, zrow, 0)

    pltpu.sync_copy(zbuf.at[pl.ds(0, EACC // NS)],
                    de_sh.at[pl.ds(tid * (EACC // NS), EACC // NS)])
    pltpu.sync_copy(zbuf, dv_sh.at[pl.ds(tid * (DVACC // NS), DVACC // NS)])
    plsc.subcore_barrier()

    def chunk(j, _):
        d1 = pltpu.async_copy(ones_v, de_sh.at[s1_v.at[j]], sem1, add=True)
        d2 = pltpu.async_copy(ones_v, dv_sh.at[s2_v.at[j]], sem2, add=True)
        d1.wait()
        d2.wait()
        return 0
    lax.fori_loop(0, NCH, chunk, 0)

    plsc.subcore_barrier()
    pltpu.sync_copy(de_sh.at[pl.ds(tid * (EACC // NS), EACC // NS)],
                    de_out.at[cid, pl.ds(tid * (EACC // NS), EACC // NS)])
    pltpu.sync_copy(dv_sh.at[pl.ds(tid * (DVACC // NS), DVACC // NS)],
                    dv_out.at[cid, pl.ds(tid * (DVACC // NS), DVACC // NS)])


def _hist(s1, s2):
    mesh = plsc.VectorSubcoreMesh(core_axis_name="c", subcore_axis_name="s")
    return pl.kernel(
        _hist_body,
        out_type=[
            jax.ShapeDtypeStruct((NC, EACC), jnp.float32),
            jax.ShapeDtypeStruct((NC, DVACC), jnp.float32),
        ],
        mesh=mesh,
        scratch_types=[
            pltpu.VMEM((NCH, 128), jnp.int32),
            pltpu.VMEM((NCH, 128), jnp.int32),
            pltpu.VMEM((128,), jnp.float32),
            pltpu.VMEM((3200,), jnp.float32),
            pltpu.VMEM_SHARED((EACC,), jnp.float32),
            pltpu.VMEM_SHARED((DVACC,), jnp.float32),
            pltpu.SemaphoreType.DMA,
            pltpu.SemaphoreType.DMA,
        ],
    )(s1, s2)


def _pad_pairs(gather_idx, scatter_idx, gather_mod, dump_base):
    pad_g = (jnp.arange(TILE_PAD, dtype=jnp.int32) * 97) % gather_mod
    pad_g = jnp.broadcast_to(pad_g[None, :], (NW, TILE_PAD))
    pad_s = dump_base + (jnp.arange(TILE_PAD, dtype=jnp.int32) % N_DUMP)
    pad_s = jnp.broadcast_to(pad_s[None, :], (NW, TILE_PAD))
    g3 = jnp.concatenate(
        [gather_idx.reshape(NW, PER_TILE), pad_g], axis=1).reshape(NW, NCH, 128)
    s3 = jnp.concatenate(
        [scatter_idx.reshape(NW, PER_TILE), pad_s], axis=1).reshape(NW, NCH, 128)
    return g3, s3


NB_IDS = B * M
PER_TILE_B = NB_IDS // NW
NCH_B = PER_TILE_B // 128
PER_TILE_C = B // NW


def _bgather_body(xp_hbm, gp_hbm, te_hbm, gid_hbm, cid_hbm,
                  xg_out, pm_out, ctx_out,
                  gid_v, cid_v, bufx, bufp, bufc, semx, semp):
    cid = lax.axis_index("c")
    tid = lax.axis_index("s")
    wid = tid * NC + cid

    pltpu.sync_copy(gid_hbm.at[wid], gid_v)
    pltpu.sync_copy(cid_hbm.at[wid], cid_v)

    pltpu.async_copy(te_hbm.at[cid_v], bufc, semx).wait()
    pltpu.sync_copy(bufc, ctx_out.at[pl.ds(wid * PER_TILE_C, PER_TILE_C)])

    obase = wid * PER_TILE_B

    def chunk(j, _):
        dx = pltpu.async_copy(xp_hbm.at[gid_v.at[j]], bufx, semx)
        dp = pltpu.async_copy(gp_hbm.at[gid_v.at[j]], bufp, semp)
        dx.wait()
        pltpu.sync_copy(bufx, xg_out.at[pl.ds(obase + j * 128, 128)])
        dp.wait()
        pltpu.sync_copy(bufp, pm_out.at[pl.ds(obase + j * 128, 128)])
        return 0
    lax.fori_loop(0, NCH_B, chunk, 0)


def _bgather(x_prop, gp_pad, treat_embed, gid3, cid2):
    mesh = plsc.VectorSubcoreMesh(core_axis_name="c", subcore_axis_name="s")
    return pl.kernel(
        _bgather_body,
        out_type=[
            jax.ShapeDtypeStruct((NB_IDS, EMBED), jnp.float32),
            jax.ShapeDtypeStruct((NB_IDS, P_PAD), jnp.float32),
            jax.ShapeDtypeStruct((B, EMBED), jnp.float32),
        ],
        mesh=mesh,
        compiler_params=pltpu.CompilerParams(use_tc_tiling_on_sc=False),
        scratch_types=[
            pltpu.VMEM((NCH_B, 128), jnp.int32),
            pltpu.VMEM((PER_TILE_C,), jnp.int32),
            pltpu.VMEM((128, EMBED), jnp.float32),
            pltpu.VMEM((128, P_PAD), jnp.float32),
            pltpu.VMEM((PER_TILE_C, EMBED), jnp.float32),
            pltpu.SemaphoreType.DMA,
            pltpu.SemaphoreType.DMA,
        ],
    )(x_prop, gp_pad, treat_embed, gid3, cid2)


def _dense_body(xg_ref, pmask_ref, ctx_ref,
                w1a_ref, w1b_ref, b1_ref, w2_ref, b2_ref,
                lw_ref, lb_ref, rw_ref, rb_ref,
                risk_ref, z_ref):
    ctx = ctx_ref[...]
    ctx_h = jnp.dot(ctx, w1b_ref[...], preferred_element_type=jnp.float32)

    xg3 = xg_ref[...]
    pm3 = pmask_ref[...]
    pgs3 = lax.dot_general(pm3, xg3, (((1,), (1,)), ((0,), (0,))),
                           preferred_element_type=jnp.float32)
    counts3 = jnp.clip(jnp.sum(pm3, axis=1), 1.0, None)

    def one_batch(b):
        pgs = pgs3[b]
        reps = pgs / counts3[b][:, None]
        h = jnp.tanh(jnp.dot(reps, w1a_ref[...],
                             preferred_element_type=jnp.float32)
                     + ctx_h[b][None, :] + b1_ref[...])
        scores = jnp.dot(h, w2_ref[...],
                         preferred_element_type=jnp.float32)[:, 0] + b2_ref[0, 0]
        pid = lax.broadcasted_iota(jnp.int32, (P_PAD,), 0)
        scores = jnp.where(pid < NUM_PATH, scores, -jnp.inf)
        scores = scores - jnp.max(scores)
        e = jnp.exp(scores)
        w = e / jnp.sum(e)
        z = jnp.dot(w[None, :], reps,
                    preferred_element_type=jnp.float32)
        z_ref[b, :] = z[0]

    for b in range(TB):
        one_batch(b)
    zlat = (jnp.dot(z_ref[...], lw_ref[...], preferred_element_type=jnp.float32)
            + lb_ref[...])
    z_ref[...] = zlat
    risk_ref[...] = (jnp.dot(zlat, rw_ref[...],
                             preferred_element_type=jnp.float32)
                     + rb_ref[0, 0])


def _dense_stage(xg, pmask, ctx, path_w1, path_b1, path_w2, path_b2,
                 latent_w, latent_b, risk_w, risk_b):
    w1a = path_w1[:EMBED]
    w1b = path_w1[EMBED:]
    grid = (B // TB,)
    flt = jnp.float32
    risk, z = pl.pallas_call(
        _dense_body,
        grid=grid,
        in_specs=[
            pl.BlockSpec((TB, M, EMBED), lambda i: (i, 0, 0)),
            pl.BlockSpec((TB, M, P_PAD), lambda i: (i, 0, 0)),
            pl.BlockSpec((TB, EMBED), lambda i: (i, 0)),
            pl.BlockSpec((EMBED, EMBED), lambda i: (0, 0)),
            pl.BlockSpec((EMBED, EMBED), lambda i: (0, 0)),
            pl.BlockSpec((EMBED,), lambda i: (0,)),
            pl.BlockSpec((EMBED, 1), lambda i: (0, 0)),
            pl.BlockSpec((1, 1), lambda i: (0, 0)),
            pl.BlockSpec((EMBED, LATENT), lambda i: (0, 0)),
            pl.BlockSpec((LATENT,), lambda i: (0,)),
            pl.BlockSpec((LATENT, 1), lambda i: (0, 0)),
            pl.BlockSpec((1, 1), lambda i: (0, 0)),
        ],
        out_specs=[
            pl.BlockSpec((TB, 1), lambda i: (i, 0)),
            pl.BlockSpec((TB, LATENT), lambda i: (i, 0)),
        ],
        out_shape=[
            jax.ShapeDtypeStruct((B, 1), flt),
            jax.ShapeDtypeStruct((B, LATENT), flt),
        ],
    )(xg, pmask, ctx, w1a, w1b, path_b1, path_w2,
      path_b2.reshape(1, 1), latent_w, latent_b, risk_w, risk_b.reshape(1, 1))
    return risk[:, 0], z


def kernel(gene_ids, context_ids, gene_embed, treat_embed, h_rows, h_cols,
           h_vals, gene_pathway, path_w1, path_b1, path_w2, path_b2,
           latent_w, latent_b, risk_w, risk_b):
    g1, s1 = _pad_pairs(h_rows, h_cols, NUM_GENES, NUM_EDGES)
    g2, s2 = _pad_pairs(h_cols, h_rows, NUM_EDGES, GDUMP)
    de_p, dv_p = _hist(s1, s2)
    Dv = dv_p[0, :NUM_GENES] + dv_p[1, :NUM_GENES]
    De = de_p[0, :NUM_EDGES] + de_p[1, :NUM_EDGES]
    Dv_inv_sqrt = jnp.power(Dv + 1e-06, -0.5)[:, None]
    De_inv = jnp.power(De + 1e-06, -1.0)[:, None]
    X = gene_embed * Dv_inv_sqrt
    xt = X.reshape(NUM_GENES, NPASS1, DSL1).transpose(1, 0, 2).reshape(
        NPASS1 * NUM_GENES, DSL1)
    hx_part = _hop1(xt, g1, s1)
    hxb = (hx_part[0] + hx_part[1])[:, :NUM_EDGES] * De_inv[None]
    hxt = hxb.reshape(NPASS1, NUM_EDGES, DSL1 // DSL, DSL).transpose(
        0, 2, 1, 3).reshape(NPASS * NUM_EDGES, DSL)
    xp_part = _hop2(hxt, g2, s2)
    xp = (xp_part[0] + xp_part[1]).transpose(1, 0, 2).reshape(GACC2, EMBED)
    X_prop = xp[:NUM_GENES] * Dv_inv_sqrt

    gp_pad = jnp.pad(gene_pathway, ((0, 0), (0, P_PAD - NUM_PATH)))
    gid3 = gene_ids.reshape(NW, NCH_B, 128)
    cid2 = context_ids.reshape(NW, PER_TILE_C)
    xg_flat, pm_flat, ctx = _bgather(X_prop, gp_pad, treat_embed, gid3, cid2)
    xg = xg_flat.reshape(B, M, EMBED)
    pmask = pm_flat.reshape(B, M, P_PAD)

    return _dense_stage(xg, pmask, ctx, path_w1, path_b1, path_w2, path_b2,
                        latent_w, latent_b, risk_w, risk_b)


# speedup vs baseline: 2.4928x; 1.2010x over previous
import functools
import jax
import jax.numpy as jnp
from jax import lax
from jax.experimental import pallas as pl
from jax.experimental.pallas import tpu as pltpu
from jax.experimental.pallas import tpu_sc as plsc

NUM_GENES = 50000
NUM_EDGES = 10000
NNZ = 500000
EMBED = 128
LATENT = 128
NUM_TREAT = 100
NUM_PATH = 50
P_PAD = 64
B = 1024
M = 200

TB = 32

NC = 2
NS = 16
NW = NC * NS
PER_TILE = NNZ // NW
NCH = 124
TILE_PAD = NCH * 128 - PER_TILE
N_DUMP = 16
EACC = 10240


def _make_prop(nacc, dsl, npass, table_mod):
    zslice = nacc // NS

    def body(tab, g_hbm, s_hbm, out_hbm,
             g_v, s_v, bufa, bufb, zbuf, acc_sh, sema, semb):
        cid = lax.axis_index("c")
        tid = lax.axis_index("s")
        wid = tid * NC + cid

        pltpu.sync_copy(g_hbm.at[wid], g_v)
        pltpu.sync_copy(s_hbm.at[wid], s_v)

        zeros = jnp.zeros((16,), jnp.float32)

        def zrow(r, _):
            for k in range(dsl // 16):
                zbuf[r, pl.ds(k * 16, 16)] = zeros
            return 0
        lax.fori_loop(0, 128, zrow, 0)

        for p in range(npass):
            if p > 0:
                def badd(j, _):
                    for k in range(8):
                        g_v[j, pl.ds(k * 16, 16)] = (
                            g_v[j, pl.ds(k * 16, 16)] + table_mod)
                    return 0
                lax.fori_loop(0, NCH, badd, 0)
            zbase = tid * zslice
            for sstep in range(zslice // 128):
                pltpu.sync_copy(zbuf, acc_sh.at[pl.ds(zbase + sstep * 128, 128)])
            rem = zslice % 128
            if rem:
                pltpu.sync_copy(
                    zbuf.at[pl.ds(0, rem)],
                    acc_sh.at[pl.ds(zbase + (zslice // 128) * 128, rem)])
            plsc.subcore_barrier()

            pltpu.async_copy(tab.at[g_v.at[0]], bufa, sema)

            def trip2(i, _):
                j0 = 2 * i
                j1 = 2 * i + 1
                d1 = pltpu.async_copy(tab.at[g_v.at[j1]], bufb, semb)
                pltpu.make_async_copy(tab.at[g_v.at[j0]], bufa, sema).wait()
                pltpu.sync_copy(bufa, acc_sh.at[s_v.at[j0]], add=True)

                @pl.when(j1 + 1 < NCH)
                def _():
                    pltpu.async_copy(tab.at[g_v.at[j1 + 1]], bufa, sema)
                d1.wait()
                pltpu.sync_copy(bufb, acc_sh.at[s_v.at[j1]], add=True)
                return 0
            lax.fori_loop(0, NCH // 2, trip2, 0)

            plsc.subcore_barrier()
            pltpu.sync_copy(acc_sh.at[pl.ds(tid * zslice, zslice)],
                            out_hbm.at[cid, p, pl.ds(tid * zslice, zslice)])
            plsc.subcore_barrier()

    mesh = plsc.VectorSubcoreMesh(core_axis_name="c", subcore_axis_name="s")

    def run(tab, g3, s3):
        return pl.kernel(
            body,
            out_type=jax.ShapeDtypeStruct((NC, npass, nacc, dsl), jnp.float32),
            mesh=mesh,
            compiler_params=pltpu.CompilerParams(use_tc_tiling_on_sc=False),
            scratch_types=[
                pltpu.VMEM((NCH, 128), jnp.int32),
                pltpu.VMEM((NCH, 128), jnp.int32),
                pltpu.VMEM((128, dsl), jnp.float32),
                pltpu.VMEM((128, dsl), jnp.float32),
                pltpu.VMEM((128, dsl), jnp.float32),
                pltpu.VMEM_SHARED((nacc, dsl), jnp.float32),
                pltpu.SemaphoreType.DMA,
                pltpu.SemaphoreType.DMA,
            ],
        )(tab, g3, s3)
    return run


GDUMP = 50176
GACC2 = 50304
DSL = 16
NPASS = EMBED // DSL
DSL1 = 64
NPASS1 = EMBED // DSL1

_hop1 = _make_prop(EACC, DSL1, NPASS1, NUM_GENES)
_hop2 = _make_prop(GACC2, DSL, NPASS, NUM_EDGES)


DVACC = 51200


def _hist_body(s1_hbm, s2_hbm, de_out, dv_out,
               s1_v, s2_v, ones_v, zbuf, de_sh, dv_sh, sem1, sem2):
    cid = lax.axis_index("c")
    tid = lax.axis_index("s")
    wid = tid * NC + cid

    pltpu.sync_copy(s1_hbm.at[wid], s1_v)
    pltpu.sync_copy(s2_hbm.at[wid], s2_v)

    ones = jnp.full((16,), 1.0, jnp.float32)
    zeros = jnp.zeros((16,), jnp.float32)
    for k in range(8):
        ones_v[pl.ds(k * 16, 16)] = ones

    def zrow(r, _):
        zbuf[pl.ds(r * 16, 16)] = zeros
        return 0
    lax.fori_loop(0, 200, zrow, 0)

    pltpu.sync_copy(zbuf.at[pl.ds(0, EACC // NS)],
                    de_sh.at[pl.ds(tid * (EACC // NS), EACC // NS)])
    pltpu.sync_copy(zbuf, dv_sh.at[pl.ds(tid * (DVACC // NS), DVACC // NS)])
    plsc.subcore_barrier()

    def chunk(j, _):
        d1 = pltpu.async_copy(ones_v, de_sh.at[s1_v.at[j]], sem1, add=True)
        d2 = pltpu.async_copy(ones_v, dv_sh.at[s2_v.at[j]], sem2, add=True)
        d1.wait()
        d2.wait()
        return 0
    lax.fori_loop(0, NCH, chunk, 0)

    plsc.subcore_barrier()
    pltpu.sync_copy(de_sh.at[pl.ds(tid * (EACC // NS), EACC // NS)],
                    de_out.at[cid, pl.ds(tid * (EACC // NS), EACC // NS)])
    pltpu.sync_copy(dv_sh.at[pl.ds(tid * (DVACC // NS), DVACC // NS)],
                    dv_out.at[cid, pl.ds(tid * (DVACC // NS), DVACC // NS)])


def _hist(s1, s2):
    mesh = plsc.VectorSubcoreMesh(core_axis_name="c", subcore_axis_name="s")
    return pl.kernel(
        _hist_body,
        out_type=[
            jax.ShapeDtypeStruct((NC, EACC), jnp.float32),
            jax.ShapeDtypeStruct((NC, DVACC), jnp.float32),
        ],
        mesh=mesh,
        scratch_types=[
            pltpu.VMEM((NCH, 128), jnp.int32),
            pltpu.VMEM((NCH, 128), jnp.int32),
            pltpu.VMEM((128,), jnp.float32),
            pltpu.VMEM((3200,), jnp.float32),
            pltpu.VMEM_SHARED((EACC,), jnp.float32),
            pltpu.VMEM_SHARED((DVACC,), jnp.float32),
            pltpu.SemaphoreType.DMA,
            pltpu.SemaphoreType.DMA,
        ],
    )(s1, s2)


def _pad_pairs(gather_idx, scatter_idx, gather_mod, dump_base):
    pad_g = (jnp.arange(TILE_PAD, dtype=jnp.int32) * 97) % gather_mod
    pad_g = jnp.broadcast_to(pad_g[None, :], (NW, TILE_PAD))
    pad_s = dump_base + (jnp.arange(TILE_PAD, dtype=jnp.int32) % N_DUMP)
    pad_s = jnp.broadcast_to(pad_s[None, :], (NW, TILE_PAD))
    g3 = jnp.concatenate(
        [gather_idx.reshape(NW, PER_TILE), pad_g], axis=1).reshape(NW, NCH, 128)
    s3 = jnp.concatenate(
        [scatter_idx.reshape(NW, PER_TILE), pad_s], axis=1).reshape(NW, NCH, 128)
    return g3, s3


NB_IDS = B * M
PER_TILE_B = NB_IDS // NW
NCH_B = PER_TILE_B // 128
PER_TILE_C = B // NW


def _bgather_body(xp_hbm, gp_hbm, te_hbm, gid_hbm, cid_hbm,
                  xg_out, pm_out, ctx_out,
                  gid_v, cid_v, bufx, bufp, bufc, semx, semp):
    cid = lax.axis_index("c")
    tid = lax.axis_index("s")
    wid = tid * NC + cid

    pltpu.sync_copy(gid_hbm.at[wid], gid_v)
    pltpu.sync_copy(cid_hbm.at[wid], cid_v)

    pltpu.async_copy(te_hbm.at[cid_v], bufc, semx).wait()
    pltpu.sync_copy(bufc, ctx_out.at[pl.ds(wid * PER_TILE_C, PER_TILE_C)])

    obase = wid * PER_TILE_B

    def chunk(j, _):
        dx = pltpu.async_copy(xp_hbm.at[gid_v.at[j]], bufx, semx)
        dp = pltpu.async_copy(gp_hbm.at[gid_v.at[j]], bufp, semp)
        dx.wait()
        pltpu.sync_copy(bufx, xg_out.at[pl.ds(obase + j * 128, 128)])
        dp.wait()
        pltpu.sync_copy(bufp, pm_out.at[pl.ds(obase + j * 128, 128)])
        return 0
    lax.fori_loop(0, NCH_B, chunk, 0)


def _bgather(x_prop, gp_pad, treat_embed, gid3, cid2):
    mesh = plsc.VectorSubcoreMesh(core_axis_name="c", subcore_axis_name="s")
    return pl.kernel(
        _bgather_body,
        out_type=[
            jax.ShapeDtypeStruct((NB_IDS, EMBED), jnp.float32),
            jax.ShapeDtypeStruct((NB_IDS, P_PAD), jnp.float32),
            jax.ShapeDtypeStruct((B, EMBED), jnp.float32),
        ],
        mesh=mesh,
        compiler_params=pltpu.CompilerParams(use_tc_tiling_on_sc=False),
        scratch_types=[
            pltpu.VMEM((NCH_B, 128), jnp.int32),
            pltpu.VMEM((PER_TILE_C,), jnp.int32),
            pltpu.VMEM((128, EMBED), jnp.float32),
            pltpu.VMEM((128, P_PAD), jnp.float32),
            pltpu.VMEM((PER_TILE_C, EMBED), jnp.float32),
            pltpu.SemaphoreType.DMA,
            pltpu.SemaphoreType.DMA,
        ],
    )(x_prop, gp_pad, treat_embed, gid3, cid2)


def _dense_body(xg_ref, pmask_ref, ctx_ref,
                w1a_ref, w1b_ref, b1_ref, w2_ref, b2_ref,
                lw_ref, lb_ref, rw_ref, rb_ref,
                risk_ref, z_ref):
    ctx = ctx_ref[...]
    ctx_h = jnp.dot(ctx, w1b_ref[...], preferred_element_type=jnp.float32)

    xg3 = xg_ref[...]
    pm3 = pmask_ref[...]
    pgs3 = lax.dot_general(pm3, xg3, (((1,), (1,)), ((0,), (0,))),
                           preferred_element_type=jnp.float32)
    counts3 = jnp.clip(jnp.sum(pm3, axis=1), 1.0, None)
    reps3 = pgs3 / counts3[:, :, None]

    flat = reps3.reshape(TB * P_PAD, EMBED)
    ctx_b = jnp.broadcast_to(ctx_h[:, None, :], (TB, P_PAD, EMBED)).reshape(
        TB * P_PAD, EMBED)
    h = jnp.tanh(jnp.dot(flat, w1a_ref[...],
                         preferred_element_type=jnp.float32) + ctx_b
                 + b1_ref[...][None, :])
    scores = (jnp.dot(h, w2_ref[...], preferred_element_type=jnp.float32)
              [:, 0] + b2_ref[0, 0]).reshape(TB, P_PAD)
    pid = lax.broadcasted_iota(jnp.int32, (TB, P_PAD), 1)
    scores = jnp.where(pid < NUM_PATH, scores, -jnp.inf)
    scores = scores - jnp.max(scores, axis=1, keepdims=True)
    e = jnp.exp(scores)
    w3 = e / jnp.sum(e, axis=1, keepdims=True)
    z_ref[...] = jnp.sum(w3[:, :, None] * reps3, axis=1)
    zlat = (jnp.dot(z_ref[...], lw_ref[...], preferred_element_type=jnp.float32)
            + lb_ref[...])
    z_ref[...] = zlat
    risk_ref[...] = (jnp.dot(zlat, rw_ref[...],
                             preferred_element_type=jnp.float32)
                     + rb_ref[0, 0])


def _dense_stage(xg, pmask, ctx, path_w1, path_b1, path_w2, path_b2,
                 latent_w, latent_b, risk_w, risk_b):
    w1a = path_w1[:EMBED]
    w1b = path_w1[EMBED:]
    grid = (B // TB,)
    flt = jnp.float32
    risk, z = pl.pallas_call(
        _dense_body,
        grid=grid,
        in_specs=[
            pl.BlockSpec((TB, M, EMBED), lambda i: (i, 0, 0)),
            pl.BlockSpec((TB, M, P_PAD), lambda i: (i, 0, 0)),
            pl.BlockSpec((TB, EMBED), lambda i: (i, 0)),
            pl.BlockSpec((EMBED, EMBED), lambda i: (0, 0)),
            pl.BlockSpec((EMBED, EMBED), lambda i: (0, 0)),
            pl.BlockSpec((EMBED,), lambda i: (0,)),
            pl.BlockSpec((EMBED, 1), lambda i: (0, 0)),
            pl.BlockSpec((1, 1), lambda i: (0, 0)),
            pl.BlockSpec((EMBED, LATENT), lambda i: (0, 0)),
            pl.BlockSpec((LATENT,), lambda i: (0,)),
            pl.BlockSpec((LATENT, 1), lambda i: (0, 0)),
            pl.BlockSpec((1, 1), lambda i: (0, 0)),
        ],
        out_specs=[
            pl.BlockSpec((TB, 1), lambda i: (i, 0)),
            pl.BlockSpec((TB, LATENT), lambda i: (i, 0)),
        ],
        out_shape=[
            jax.ShapeDtypeStruct((B, 1), flt),
            jax.ShapeDtypeStruct((B, LATENT), flt),
        ],
    )(xg, pmask, ctx, w1a, w1b, path_b1, path_w2,
      path_b2.reshape(1, 1), latent_w, latent_b, risk_w, risk_b.reshape(1, 1))
    return risk[:, 0], z


def kernel(gene_ids, context_ids, gene_embed, treat_embed, h_rows, h_cols,
           h_vals, gene_pathway, path_w1, path_b1, path_w2, path_b2,
           latent_w, latent_b, risk_w, risk_b):
    g1, s1 = _pad_pairs(h_rows, h_cols, NUM_GENES, NUM_EDGES)
    g2, s2 = _pad_pairs(h_cols, h_rows, NUM_EDGES, GDUMP)
    de_p, dv_p = _hist(s1, s2)
    Dv = dv_p[0, :NUM_GENES] + dv_p[1, :NUM_GENES]
    De = de_p[0, :NUM_EDGES] + de_p[1, :NUM_EDGES]
    Dv_inv_sqrt = jnp.power(Dv + 1e-06, -0.5)[:, None]
    De_inv = jnp.power(De + 1e-06, -1.0)[:, None]
    X = gene_embed * Dv_inv_sqrt
    xt = X.reshape(NUM_GENES, NPASS1, DSL1).transpose(1, 0, 2).reshape(
        NPASS1 * NUM_GENES, DSL1)
    hx_part = _hop1(xt, g1, s1)
    hxb = (hx_part[0] + hx_part[1])[:, :NUM_EDGES] * De_inv[None]
    hxt = hxb.reshape(NPASS1, NUM_EDGES, DSL1 // DSL, DSL).transpose(
        0, 2, 1, 3).reshape(NPASS * NUM_EDGES, DSL)
    xp_part = _hop2(hxt, g2, s2)
    xp = (xp_part[0] + xp_part[1]).transpose(1, 0, 2).reshape(GACC2, EMBED)
    X_prop = xp[:NUM_GENES] * Dv_inv_sqrt

    gp_pad = jnp.pad(gene_pathway, ((0, 0), (0, P_PAD - NUM_PATH)))
    gid3 = gene_ids.reshape(NW, NCH_B, 128)
    cid2 = context_ids.reshape(NW, PER_TILE_C)
    xg_flat, pm_flat, ctx = _bgather(X_prop, gp_pad, treat_embed, gid3, cid2)
    xg = xg_flat.reshape(B, M, EMBED)
    pmask = pm_flat.reshape(B, M, P_PAD)

    return _dense_stage(xg, pmask, ctx, path_w1, path_b1, path_w2, path_b2,
                        latent_w, latent_b, risk_w, risk_b)


# speedup vs baseline: 2.5041x; 1.0045x over previous
import functools
import jax
import jax.numpy as jnp
from jax import lax
from jax.experimental import pallas as pl
from jax.experimental.pallas import tpu as pltpu
from jax.experimental.pallas import tpu_sc as plsc

NUM_GENES = 50000
NUM_EDGES = 10000
NNZ = 500000
EMBED = 128
LATENT = 128
NUM_TREAT = 100
NUM_PATH = 50
P_PAD = 64
B = 1024
M = 200

TB = 128

NC = 2
NS = 16
NW = NC * NS
PER_TILE = NNZ // NW
NCH = 124
TILE_PAD = NCH * 128 - PER_TILE
N_DUMP = 16
EACC = 10240


def _make_prop(nacc, dsl, npass, table_mod):
    zslice = nacc // NS

    def body(tab, g_hbm, s_hbm, out_hbm,
             g_v, s_v, bufa, bufb, zbuf, acc_sh, sema, semb):
        cid = lax.axis_index("c")
        tid = lax.axis_index("s")
        wid = tid * NC + cid

        pltpu.sync_copy(g_hbm.at[wid], g_v)
        pltpu.sync_copy(s_hbm.at[wid], s_v)

        zeros = jnp.zeros((16,), jnp.float32)

        def zrow(r, _):
            for k in range(dsl // 16):
                zbuf[r, pl.ds(k * 16, 16)] = zeros
            return 0
        lax.fori_loop(0, 128, zrow, 0)

        for p in range(npass):
            if p > 0:
                def badd(j, _):
                    for k in range(8):
                        g_v[j, pl.ds(k * 16, 16)] = (
                            g_v[j, pl.ds(k * 16, 16)] + table_mod)
                    return 0
                lax.fori_loop(0, NCH, badd, 0)
            zbase = tid * zslice
            for sstep in range(zslice // 128):
                pltpu.sync_copy(zbuf, acc_sh.at[pl.ds(zbase + sstep * 128, 128)])
            rem = zslice % 128
            if rem:
                pltpu.sync_copy(
                    zbuf.at[pl.ds(0, rem)],
                    acc_sh.at[pl.ds(zbase + (zslice // 128) * 128, rem)])
            plsc.subcore_barrier()

            pltpu.async_copy(tab.at[g_v.at[0]], bufa, sema)

            def trip2(i, _):
                j0 = 2 * i
                j1 = 2 * i + 1
                d1 = pltpu.async_copy(tab.at[g_v.at[j1]], bufb, semb)
                pltpu.make_async_copy(tab.at[g_v.at[j0]], bufa, sema).wait()
                pltpu.sync_copy(bufa, acc_sh.at[s_v.at[j0]], add=True)

                @pl.when(j1 + 1 < NCH)
                def _():
                    pltpu.async_copy(tab.at[g_v.at[j1 + 1]], bufa, sema)
                d1.wait()
                pltpu.sync_copy(bufb, acc_sh.at[s_v.at[j1]], add=True)
                return 0
            lax.fori_loop(0, NCH // 2, trip2, 0)

            plsc.subcore_barrier()
            pltpu.sync_copy(acc_sh.at[pl.ds(tid * zslice, zslice)],
                            out_hbm.at[cid, p, pl.ds(tid * zslice, zslice)])
            plsc.subcore_barrier()

    mesh = plsc.VectorSubcoreMesh(core_axis_name="c", subcore_axis_name="s")

    def run(tab, g3, s3):
        return pl.kernel(
            body,
            out_type=jax.ShapeDtypeStruct((NC, npass, nacc, dsl), jnp.float32),
            mesh=mesh,
            compiler_params=pltpu.CompilerParams(use_tc_tiling_on_sc=False),
            scratch_types=[
                pltpu.VMEM((NCH, 128), jnp.int32),
                pltpu.VMEM((NCH, 128), jnp.int32),
                pltpu.VMEM((128, dsl), jnp.float32),
                pltpu.VMEM((128, dsl), jnp.float32),
                pltpu.VMEM((128, dsl), jnp.float32),
                pltpu.VMEM_SHARED((nacc, dsl), jnp.float32),
                pltpu.SemaphoreType.DMA,
                pltpu.SemaphoreType.DMA,
            ],
        )(tab, g3, s3)
    return run


GDUMP = 50176
GACC2 = 50304
DSL = 16
NPASS = EMBED // DSL
DSL1 = 64
NPASS1 = EMBED // DSL1

_hop1 = _make_prop(EACC, DSL1, NPASS1, NUM_GENES)
_hop2 = _make_prop(GACC2, DSL, NPASS, NUM_EDGES)


DVACC = 51200


def _hist_body(s1_hbm, s2_hbm, de_out, dv_out,
               s1_v, s2_v, ones_v, zbuf, de_sh, dv_sh, sem1, sem2):
    cid = lax.axis_index("c")
    tid = lax.axis_index("s")
    wid = tid * NC + cid

    pltpu.sync_copy(s1_hbm.at[wid], s1_v)
    pltpu.sync_copy(s2_hbm.at[wid], s2_v)

    ones = jnp.full((16,), 1.0, jnp.float32)
    zeros = jnp.zeros((16,), jnp.float32)
    for k in range(8):
        ones_v[pl.ds(k * 16, 16)] = ones

    def zrow(r, _):
        zbuf[pl.ds(r * 16, 16)] = zeros
        return 0
    lax.fori_loop(0, 200, zrow, 0)

    pltpu.sync_copy(zbuf.at[pl.ds(0, EACC // NS)],
                    de_sh.at[pl.ds(tid * (EACC // NS), EACC // NS)])
    pltpu.sync_copy(zbuf, dv_sh.at[pl.ds(tid * (DVACC // NS), DVACC // NS)])
    plsc.subcore_barrier()

    def chunk(j, _):
        d1 = pltpu.async_copy(ones_v, de_sh.at[s1_v.at[j]], sem1, add=True)
        d2 = pltpu.async_copy(ones_v, dv_sh.at[s2_v.at[j]], sem2, add=True)
        d1.wait()
        d2.wait()
        return 0
    lax.fori_loop(0, NCH, chunk, 0)

    plsc.subcore_barrier()
    pltpu.sync_copy(de_sh.at[pl.ds(tid * (EACC // NS), EACC // NS)],
                    de_out.at[cid, pl.ds(tid * (EACC // NS), EACC // NS)])
    pltpu.sync_copy(dv_sh.at[pl.ds(tid * (DVACC // NS), DVACC // NS)],
                    dv_out.at[cid, pl.ds(tid * (DVACC // NS), DVACC // NS)])


def _hist(s1, s2):
    mesh = plsc.VectorSubcoreMesh(core_axis_name="c", subcore_axis_name="s")
    return pl.kernel(
        _hist_body,
        out_type=[
            jax.ShapeDtypeStruct((NC, EACC), jnp.float32),
            jax.ShapeDtypeStruct((NC, DVACC), jnp.float32),
        ],
        mesh=mesh,
        scratch_types=[
            pltpu.VMEM((NCH, 128), jnp.int32),
            pltpu.VMEM((NCH, 128), jnp.int32),
            pltpu.VMEM((128,), jnp.float32),
            pltpu.VMEM((3200,), jnp.float32),
            pltpu.VMEM_SHARED((EACC,), jnp.float32),
            pltpu.VMEM_SHARED((DVACC,), jnp.float32),
            pltpu.SemaphoreType.DMA,
            pltpu.SemaphoreType.DMA,
        ],
    )(s1, s2)


def _pad_pairs(gather_idx, scatter_idx, gather_mod, dump_base):
    pad_g = (jnp.arange(TILE_PAD, dtype=jnp.int32) * 97) % gather_mod
    pad_g = jnp.broadcast_to(pad_g[None, :], (NW, TILE_PAD))
    pad_s = dump_base + (jnp.arange(TILE_PAD, dtype=jnp.int32) % N_DUMP)
    pad_s = jnp.broadcast_to(pad_s[None, :], (NW, TILE_PAD))
    g3 = jnp.concatenate(
        [gather_idx.reshape(NW, PER_TILE), pad_g], axis=1).reshape(NW, NCH, 128)
    s3 = jnp.concatenate(
        [scatter_idx.reshape(NW, PER_TILE), pad_s], axis=1).reshape(NW, NCH, 128)
    return g3, s3


NB_IDS = B * M
PER_TILE_B = NB_IDS // NW
NCH_B = PER_TILE_B // 128
PER_TILE_C = B // NW


def _bgather_body(xp_hbm, gp_hbm, te_hbm, gid_hbm, cid_hbm,
                  xg_out, pm_out, ctx_out,
                  gid_v, cid_v, bufx, bufp, bufc, semx, semp):
    cid = lax.axis_index("c")
    tid = lax.axis_index("s")
    wid = tid * NC + cid

    pltpu.sync_copy(gid_hbm.at[wid], gid_v)
    pltpu.sync_copy(cid_hbm.at[wid], cid_v)

    pltpu.async_copy(te_hbm.at[cid_v], bufc, semx).wait()
    pltpu.sync_copy(bufc, ctx_out.at[pl.ds(wid * PER_TILE_C, PER_TILE_C)])

    obase = wid * PER_TILE_B

    def chunk(j, _):
        dx = pltpu.async_copy(xp_hbm.at[gid_v.at[j]], bufx, semx)
        dp = pltpu.async_copy(gp_hbm.at[gid_v.at[j]], bufp, semp)
        dx.wait()
        pltpu.sync_copy(bufx, xg_out.at[pl.ds(obase + j * 128, 128)])
        dp.wait()
        pltpu.sync_copy(bufp, pm_out.at[pl.ds(obase + j * 128, 128)])
        return 0
    lax.fori_loop(0, NCH_B, chunk, 0)


def _bgather(x_prop, gp_pad, treat_embed, gid3, cid2):
    mesh = plsc.VectorSubcoreMesh(core_axis_name="c", subcore_axis_name="s")
    return pl.kernel(
        _bgather_body,
        out_type=[
            jax.ShapeDtypeStruct((NB_IDS, EMBED), jnp.float32),
            jax.ShapeDtypeStruct((NB_IDS, P_PAD), jnp.float32),
            jax.ShapeDtypeStruct((B, EMBED), jnp.float32),
        ],
        mesh=mesh,
        compiler_params=pltpu.CompilerParams(use_tc_tiling_on_sc=False),
        scratch_types=[
            pltpu.VMEM((NCH_B, 128), jnp.int32),
            pltpu.VMEM((PER_TILE_C,), jnp.int32),
            pltpu.VMEM((128, EMBED), jnp.float32),
            pltpu.VMEM((128, P_PAD), jnp.float32),
            pltpu.VMEM((PER_TILE_C, EMBED), jnp.float32),
            pltpu.SemaphoreType.DMA,
            pltpu.SemaphoreType.DMA,
        ],
    )(x_prop, gp_pad, treat_embed, gid3, cid2)


def _dense_body(xg_ref, pmask_ref, ctx_ref,
                w1a_ref, w1b_ref, b1_ref, w2_ref, b2_ref,
                lw_ref, lb_ref, rw_ref, rb_ref,
                risk_ref, z_ref):
    ctx = ctx_ref[...]
    ctx_h = jnp.dot(ctx, w1b_ref[...], preferred_element_type=jnp.float32)

    xg3 = xg_ref[...]
    pm3 = pmask_ref[...]
    pgs3 = lax.dot_general(pm3, xg3, (((1,), (1,)), ((0,), (0,))),
                           preferred_element_type=jnp.float32)
    counts3 = jnp.clip(jnp.sum(pm3, axis=1), 1.0, None)
    reps3 = pgs3 / counts3[:, :, None]

    flat = reps3.reshape(TB * P_PAD, EMBED)
    ctx_b = jnp.broadcast_to(ctx_h[:, None, :], (TB, P_PAD, EMBED)).reshape(
        TB * P_PAD, EMBED)
    h = jnp.tanh(jnp.dot(flat, w1a_ref[...],
                         preferred_element_type=jnp.float32) + ctx_b
                 + b1_ref[...][None, :])
    scores = (jnp.dot(h, w2_ref[...], preferred_element_type=jnp.float32)
              [:, 0] + b2_ref[0, 0]).reshape(TB, P_PAD)
    pid = lax.broadcasted_iota(jnp.int32, (TB, P_PAD), 1)
    scores = jnp.where(pid < NUM_PATH, scores, -jnp.inf)
    scores = scores - jnp.max(scores, axis=1, keepdims=True)
    e = jnp.exp(scores)
    w3 = e / jnp.sum(e, axis=1, keepdims=True)
    z_ref[...] = jnp.sum(w3[:, :, None] * reps3, axis=1)
    zlat = (jnp.dot(z_ref[...], lw_ref[...], preferred_element_type=jnp.float32)
            + lb_ref[...])
    z_ref[...] = zlat
    risk_ref[...] = (jnp.dot(zlat, rw_ref[...],
                             preferred_element_type=jnp.float32)
                     + rb_ref[0, 0])


def _dense_stage(xg, pmask, ctx, path_w1, path_b1, path_w2, path_b2,
                 latent_w, latent_b, risk_w, risk_b):
    w1a = path_w1[:EMBED]
    w1b = path_w1[EMBED:]
    grid = (B // TB,)
    flt = jnp.float32
    risk, z = pl.pallas_call(
        _dense_body,
        grid=grid,
        in_specs=[
            pl.BlockSpec((TB, M, EMBED), lambda i: (i, 0, 0)),
            pl.BlockSpec((TB, M, P_PAD), lambda i: (i, 0, 0)),
            pl.BlockSpec((TB, EMBED), lambda i: (i, 0)),
            pl.BlockSpec((EMBED, EMBED), lambda i: (0, 0)),
            pl.BlockSpec((EMBED, EMBED), lambda i: (0, 0)),
            pl.BlockSpec((EMBED,), lambda i: (0,)),
            pl.BlockSpec((EMBED, 1), lambda i: (0, 0)),
            pl.BlockSpec((1, 1), lambda i: (0, 0)),
            pl.BlockSpec((EMBED, LATENT), lambda i: (0, 0)),
            pl.BlockSpec((LATENT,), lambda i: (0,)),
            pl.BlockSpec((LATENT, 1), lambda i: (0, 0)),
            pl.BlockSpec((1, 1), lambda i: (0, 0)),
        ],
        out_specs=[
            pl.BlockSpec((TB, 1), lambda i: (i, 0)),
            pl.BlockSpec((TB, LATENT), lambda i: (i, 0)),
        ],
        out_shape=[
            jax.ShapeDtypeStruct((B, 1), flt),
            jax.ShapeDtypeStruct((B, LATENT), flt),
        ],
    )(xg, pmask, ctx, w1a, w1b, path_b1, path_w2,
      path_b2.reshape(1, 1), latent_w, latent_b, risk_w, risk_b.reshape(1, 1))
    return risk[:, 0], z


def kernel(gene_ids, context_ids, gene_embed, treat_embed, h_rows, h_cols,
           h_vals, gene_pathway, path_w1, path_b1, path_w2, path_b2,
           latent_w, latent_b, risk_w, risk_b):
    g1, s1 = _pad_pairs(h_rows, h_cols, NUM_GENES, NUM_EDGES)
    g2, s2 = _pad_pairs(h_cols, h_rows, NUM_EDGES, GDUMP)
    de_p, dv_p = _hist(s1, s2)
    Dv = dv_p[0, :NUM_GENES] + dv_p[1, :NUM_GENES]
    De = de_p[0, :NUM_EDGES] + de_p[1, :NUM_EDGES]
    Dv_inv_sqrt = jnp.power(Dv + 1e-06, -0.5)[:, None]
    De_inv = jnp.power(De + 1e-06, -1.0)[:, None]
    X = gene_embed * Dv_inv_sqrt
    xt = X.reshape(NUM_GENES, NPASS1, DSL1).transpose(1, 0, 2).reshape(
        NPASS1 * NUM_GENES, DSL1)
    hx_part = _hop1(xt, g1, s1)
    hxb = (hx_part[0] + hx_part[1])[:, :NUM_EDGES] * De_inv[None]
    hxt = hxb.reshape(NPASS1, NUM_EDGES, DSL1 // DSL, DSL).transpose(
        0, 2, 1, 3).reshape(NPASS * NUM_EDGES, DSL)
    xp_part = _hop2(hxt, g2, s2)
    xp = (xp_part[0] + xp_part[1]).transpose(1, 0, 2).reshape(GACC2, EMBED)
    X_prop = xp[:NUM_GENES] * Dv_inv_sqrt

    gp_pad = jnp.pad(gene_pathway, ((0, 0), (0, P_PAD - NUM_PATH)))
    gid3 = gene_ids.reshape(NW, NCH_B, 128)
    cid2 = context_ids.reshape(NW, PER_TILE_C)
    xg_flat, pm_flat, ctx = _bgather(X_prop, gp_pad, treat_embed, gid3, cid2)
    xg = xg_flat.reshape(B, M, EMBED)
    pmask = pm_flat.reshape(B, M, P_PAD)

    return _dense_stage(xg, pmask, ctx, path_w1, path_b1, path_w2, path_b2,
                        latent_w, latent_b, risk_w, risk_b)


# speedup vs baseline: 2.7655x; 1.1044x over previous
import functools
import jax
import jax.numpy as jnp
from jax import lax
from jax.experimental import pallas as pl
from jax.experimental.pallas import tpu as pltpu
from jax.experimental.pallas import tpu_sc as plsc

NUM_GENES = 50000
NUM_EDGES = 10000
NNZ = 500000
EMBED = 128
LATENT = 128
NUM_TREAT = 100
NUM_PATH = 50
P_PAD = 64
B = 1024
M = 200

TB = 128

NC = 2
NS = 16
NW = NC * NS
NSLOT = 4
PER_TILE = NNZ // NW
NCH = 128
TILE_PAD = NCH * 128 - PER_TILE
N_DUMP = 16
EACC = 10240


def _make_prop(nacc, dsl, npass, table_mod):
    zslice = nacc // NS

    def body(tab, g_hbm, s_hbm, out_hbm,
             g_v, s_v, zbuf, acc_sh, *bs):
        bufs = list(bs[:NSLOT])
        gs = list(bs[NSLOT:2 * NSLOT])
        ss = list(bs[2 * NSLOT:3 * NSLOT])
        cid = lax.axis_index("c")
        tid = lax.axis_index("s")
        wid = tid * NC + cid

        pltpu.sync_copy(g_hbm.at[wid], g_v)
        pltpu.sync_copy(s_hbm.at[wid], s_v)

        zeros = jnp.zeros((16,), jnp.float32)

        def zrow(r, _):
            for k in range(dsl // 16):
                zbuf[r, pl.ds(k * 16, 16)] = zeros
            return 0
        lax.fori_loop(0, 128, zrow, 0)

        for p in range(npass):
            if p > 0:
                def badd(j, _):
                    for k in range(8):
                        g_v[j, pl.ds(k * 16, 16)] = (
                            g_v[j, pl.ds(k * 16, 16)] + table_mod)
                    return 0
                lax.fori_loop(0, NCH, badd, 0)
            zbase = tid * zslice
            for sstep in range(zslice // 128):
                pltpu.sync_copy(zbuf, acc_sh.at[pl.ds(zbase + sstep * 128, 128)])
            rem = zslice % 128
            if rem:
                pltpu.sync_copy(
                    zbuf.at[pl.ds(0, rem)],
                    acc_sh.at[pl.ds(zbase + (zslice // 128) * 128, rem)])
            plsc.subcore_barrier()

            for k in range(NSLOT):
                pltpu.async_copy(tab.at[g_v.at[k]], bufs[k], gs[k])

            def piped(i, _):
                base = i * NSLOT
                for k in range(NSLOT):
                    j = base + k
                    pltpu.make_async_copy(tab.at[g_v.at[j]],
                                          bufs[k], gs[k]).wait()
                    pltpu.async_copy(bufs[k], acc_sh.at[s_v.at[j]],
                                     ss[k], add=True)

                def refill(k, j, jn):
                    @pl.when(jn < NCH)
                    def _():
                        pltpu.make_async_copy(bufs[k], acc_sh.at[s_v.at[j]],
                                              ss[k]).wait()
                        pltpu.async_copy(tab.at[g_v.at[jn]], bufs[k], gs[k])
                for k in range(NSLOT):
                    refill(k, base + k, base + k + NSLOT)
                return 0
            lax.fori_loop(0, NCH // NSLOT, piped, 0)
            for k in range(NSLOT):
                jd = NCH - NSLOT + k
                pltpu.make_async_copy(bufs[k], acc_sh.at[s_v.at[jd]],
                                      ss[k]).wait()

            plsc.subcore_barrier()
            pltpu.sync_copy(acc_sh.at[pl.ds(tid * zslice, zslice)],
                            out_hbm.at[cid, p, pl.ds(tid * zslice, zslice)])
            plsc.subcore_barrier()

    mesh = plsc.VectorSubcoreMesh(core_axis_name="c", subcore_axis_name="s")

    def run(tab, g3, s3):
        return pl.kernel(
            body,
            out_type=jax.ShapeDtypeStruct((NC, npass, nacc, dsl), jnp.float32),
            mesh=mesh,
            compiler_params=pltpu.CompilerParams(use_tc_tiling_on_sc=False),
            scratch_types=(
                [pltpu.VMEM((NCH, 128), jnp.int32),
                 pltpu.VMEM((NCH, 128), jnp.int32),
                 pltpu.VMEM((128, dsl), jnp.float32),
                 pltpu.VMEM_SHARED((nacc, dsl), jnp.float32)]
                + [pltpu.VMEM((128, dsl), jnp.float32)] * NSLOT
                + [pltpu.SemaphoreType.DMA] * (2 * NSLOT)
            ),
        )(tab, g3, s3)
    return run


GDUMP = 50176
GACC2 = 50304
DSL = 16
NPASS = EMBED // DSL
DSL1 = 64
NPASS1 = EMBED // DSL1

_hop1 = _make_prop(EACC, DSL1, NPASS1, NUM_GENES)
_hop2 = _make_prop(GACC2, DSL, NPASS, NUM_EDGES)


DVACC = 51200


def _hist_body(s1_hbm, s2_hbm, de_out, dv_out,
               s1_v, s2_v, ones_v, zbuf, de_sh, dv_sh, sem1, sem2):
    cid = lax.axis_index("c")
    tid = lax.axis_index("s")
    wid = tid * NC + cid

    pltpu.sync_copy(s1_hbm.at[wid], s1_v)
    pltpu.sync_copy(s2_hbm.at[wid], s2_v)

    ones = jnp.full((16,), 1.0, jnp.float32)
    zeros = jnp.zeros((16,), jnp.float32)
    for k in range(8):
        ones_v[pl.ds(k * 16, 16)] = ones

    def zrow(r, _):
        zbuf[pl.ds(r * 16, 16)] = zeros
        return 0
    lax.fori_loop(0, 200, zrow, 0)

    pltpu.sync_copy(zbuf.at[pl.ds(0, EACC // NS)],
                    de_sh.at[pl.ds(tid * (EACC // NS), EACC // NS)])
    pltpu.sync_copy(zbuf, dv_sh.at[pl.ds(tid * (DVACC // NS), DVACC // NS)])
    plsc.subcore_barrier()

    def chunk(j, _):
        d1 = pltpu.async_copy(ones_v, de_sh.at[s1_v.at[j]], sem1, add=True)
        d2 = pltpu.async_copy(ones_v, dv_sh.at[s2_v.at[j]], sem2, add=True)
        d1.wait()
        d2.wait()
        return 0
    lax.fori_loop(0, NCH, chunk, 0)

    plsc.subcore_barrier()
    pltpu.sync_copy(de_sh.at[pl.ds(tid * (EACC // NS), EACC // NS)],
                    de_out.at[cid, pl.ds(tid * (EACC // NS), EACC // NS)])
    pltpu.sync_copy(dv_sh.at[pl.ds(tid * (DVACC // NS), DVACC // NS)],
                    dv_out.at[cid, pl.ds(tid * (DVACC // NS), DVACC // NS)])


def _hist(s1, s2):
    mesh = plsc.VectorSubcoreMesh(core_axis_name="c", subcore_axis_name="s")
    return pl.kernel(
        _hist_body,
        out_type=[
            jax.ShapeDtypeStruct((NC, EACC), jnp.float32),
            jax.ShapeDtypeStruct((NC, DVACC), jnp.float32),
        ],
        mesh=mesh,
        scratch_types=[
            pltpu.VMEM((NCH, 128), jnp.int32),
            pltpu.VMEM((NCH, 128), jnp.int32),
            pltpu.VMEM((128,), jnp.float32),
            pltpu.VMEM((3200,), jnp.float32),
            pltpu.VMEM_SHARED((EACC,), jnp.float32),
            pltpu.VMEM_SHARED((DVACC,), jnp.float32),
            pltpu.SemaphoreType.DMA,
            pltpu.SemaphoreType.DMA,
        ],
    )(s1, s2)


def _pad_pairs(gather_idx, scatter_idx, gather_mod, dump_base):
    pad_g = (jnp.arange(TILE_PAD, dtype=jnp.int32) * 97) % gather_mod
    pad_g = jnp.broadcast_to(pad_g[None, :], (NW, TILE_PAD))
    pad_s = dump_base + (jnp.arange(TILE_PAD, dtype=jnp.int32) % N_DUMP)
    pad_s = jnp.broadcast_to(pad_s[None, :], (NW, TILE_PAD))
    g3 = jnp.concatenate(
        [gather_idx.reshape(NW, PER_TILE), pad_g], axis=1).reshape(NW, NCH, 128)
    s3 = jnp.concatenate(
        [scatter_idx.reshape(NW, PER_TILE), pad_s], axis=1).reshape(NW, NCH, 128)
    return g3, s3


NB_IDS = B * M
PER_TILE_B = NB_IDS // NW
NCH_B = PER_TILE_B // 128
PER_TILE_C = B // NW


def _bgather_body(xp_hbm, gp_hbm, te_hbm, gid_hbm, cid_hbm,
                  xg_out, pm_out, ctx_out,
                  gid_v, cid_v, bufx, bufp, bufc, semx, semp):
    cid = lax.axis_index("c")
    tid = lax.axis_index("s")
    wid = tid * NC + cid

    pltpu.sync_copy(gid_hbm.at[wid], gid_v)
    pltpu.sync_copy(cid_hbm.at[wid], cid_v)

    pltpu.async_copy(te_hbm.at[cid_v], bufc, semx).wait()
    pltpu.sync_copy(bufc, ctx_out.at[pl.ds(wid * PER_TILE_C, PER_TILE_C)])

    obase = wid * PER_TILE_B

    def chunk(j, _):
        dx = pltpu.async_copy(xp_hbm.at[gid_v.at[j]], bufx, semx)
        dp = pltpu.async_copy(gp_hbm.at[gid_v.at[j]], bufp, semp)
        dx.wait()
        pltpu.sync_copy(bufx, xg_out.at[pl.ds(obase + j * 128, 128)])
        dp.wait()
        pltpu.sync_copy(bufp, pm_out.at[pl.ds(obase + j * 128, 128)])
        return 0
    lax.fori_loop(0, NCH_B, chunk, 0)


def _bgather(x_prop, gp_pad, treat_embed, gid3, cid2):
    mesh = plsc.VectorSubcoreMesh(core_axis_name="c", subcore_axis_name="s")
    return pl.kernel(
        _bgather_body,
        out_type=[
            jax.ShapeDtypeStruct((NB_IDS, EMBED), jnp.float32),
            jax.ShapeDtypeStruct((NB_IDS, P_PAD), jnp.float32),
            jax.ShapeDtypeStruct((B, EMBED), jnp.float32),
        ],
        mesh=mesh,
        compiler_params=pltpu.CompilerParams(use_tc_tiling_on_sc=False),
        scratch_types=[
            pltpu.VMEM((NCH_B, 128), jnp.int32),
            pltpu.VMEM((PER_TILE_C,), jnp.int32),
            pltpu.VMEM((128, EMBED), jnp.float32),
            pltpu.VMEM((128, P_PAD), jnp.float32),
            pltpu.VMEM((PER_TILE_C, EMBED), jnp.float32),
            pltpu.SemaphoreType.DMA,
            pltpu.SemaphoreType.DMA,
        ],
    )(x_prop, gp_pad, treat_embed, gid3, cid2)


def _dense_body(xg_ref, pmask_ref, ctx_ref,
                w1a_ref, w1b_ref, b1_ref, w2_ref, b2_ref,
                lw_ref, lb_ref, rw_ref, rb_ref,
                risk_ref, z_ref):
    ctx = ctx_ref[...]
    ctx_h = jnp.dot(ctx, w1b_ref[...], preferred_element_type=jnp.float32)

    xg3 = xg_ref[...]
    pm3 = pmask_ref[...]
    pgs3 = lax.dot_general(pm3, xg3, (((1,), (1,)), ((0,), (0,))),
                           preferred_element_type=jnp.float32)
    counts3 = jnp.clip(jnp.sum(pm3, axis=1), 1.0, None)
    reps3 = pgs3 / counts3[:, :, None]

    flat = reps3.reshape(TB * P_PAD, EMBED)
    ctx_b = jnp.broadcast_to(ctx_h[:, None, :], (TB, P_PAD, EMBED)).reshape(
        TB * P_PAD, EMBED)
    h = jnp.tanh(jnp.dot(flat, w1a_ref[...],
                         preferred_element_type=jnp.float32) + ctx_b
                 + b1_ref[...][None, :])
    scores = (jnp.dot(h, w2_ref[...], preferred_element_type=jnp.float32)
              [:, 0] + b2_ref[0, 0]).reshape(TB, P_PAD)
    pid = lax.broadcasted_iota(jnp.int32, (TB, P_PAD), 1)
    scores = jnp.where(pid < NUM_PATH, scores, -jnp.inf)
    scores = scores - jnp.max(scores, axis=1, keepdims=True)
    e = jnp.exp(scores)
    w3 = e / jnp.sum(e, axis=1, keepdims=True)
    z_ref[...] = jnp.sum(w3[:, :, None] * reps3, axis=1)
    zlat = (jnp.dot(z_ref[...], lw_ref[...], preferred_element_type=jnp.float32)
            + lb_ref[...])
    z_ref[...] = zlat
    risk_ref[...] = (jnp.dot(zlat, rw_ref[...],
                             preferred_element_type=jnp.float32)
                     + rb_ref[0, 0])


def _dense_stage(xg, pmask, ctx, path_w1, path_b1, path_w2, path_b2,
                 latent_w, latent_b, risk_w, risk_b):
    w1a = path_w1[:EMBED]
    w1b = path_w1[EMBED:]
    grid = (B // TB,)
    flt = jnp.float32
    risk, z = pl.pallas_call(
        _dense_body,
        grid=grid,
        in_specs=[
            pl.BlockSpec((TB, M, EMBED), lambda i: (i, 0, 0)),
            pl.BlockSpec((TB, M, P_PAD), lambda i: (i, 0, 0)),
            pl.BlockSpec((TB, EMBED), lambda i: (i, 0)),
            pl.BlockSpec((EMBED, EMBED), lambda i: (0, 0)),
            pl.BlockSpec((EMBED, EMBED), lambda i: (0, 0)),
            pl.BlockSpec((EMBED,), lambda i: (0,)),
            pl.BlockSpec((EMBED, 1), lambda i: (0, 0)),
            pl.BlockSpec((1, 1), lambda i: (0, 0)),
            pl.BlockSpec((EMBED, LATENT), lambda i: (0, 0)),
            pl.BlockSpec((LATENT,), lambda i: (0,)),
            pl.BlockSpec((LATENT, 1), lambda i: (0, 0)),
            pl.BlockSpec((1, 1), lambda i: (0, 0)),
        ],
        out_specs=[
            pl.BlockSpec((TB, 1), lambda i: (i, 0)),
            pl.BlockSpec((TB, LATENT), lambda i: (i, 0)),
        ],
        out_shape=[
            jax.ShapeDtypeStruct((B, 1), flt),
            jax.ShapeDtypeStruct((B, LATENT), flt),
        ],
    )(xg, pmask, ctx, w1a, w1b, path_b1, path_w2,
      path_b2.reshape(1, 1), latent_w, latent_b, risk_w, risk_b.reshape(1, 1))
    return risk[:, 0], z


def kernel(gene_ids, context_ids, gene_embed, treat_embed, h_rows, h_cols,
           h_vals, gene_pathway, path_w1, path_b1, path_w2, path_b2,
           latent_w, latent_b, risk_w, risk_b):
    g1, s1 = _pad_pairs(h_rows, h_cols, NUM_GENES, NUM_EDGES)
    g2, s2 = _pad_pairs(h_cols, h_rows, NUM_EDGES, GDUMP)
    de_p, dv_p = _hist(s1, s2)
    Dv = dv_p[0, :NUM_GENES] + dv_p[1, :NUM_GENES]
    De = de_p[0, :NUM_EDGES] + de_p[1, :NUM_EDGES]
    Dv_inv_sqrt = jnp.power(Dv + 1e-06, -0.5)[:, None]
    De_inv = jnp.power(De + 1e-06, -1.0)[:, None]
    X = gene_embed * Dv_inv_sqrt
    xt = X.reshape(NUM_GENES, NPASS1, DSL1).transpose(1, 0, 2).reshape(
        NPASS1 * NUM_GENES, DSL1)
    hx_part = _hop1(xt, g1, s1)
    hxb = (hx_part[0] + hx_part[1])[:, :NUM_EDGES] * De_inv[None]
    hxt = hxb.reshape(NPASS1, NUM_EDGES, DSL1 // DSL, DSL).transpose(
        0, 2, 1, 3).reshape(NPASS * NUM_EDGES, DSL)
    xp_part = _hop2(hxt, g2, s2)
    xp = (xp_part[0] + xp_part[1]).transpose(1, 0, 2).reshape(GACC2, EMBED)
    X_prop = xp[:NUM_GENES] * Dv_inv_sqrt

    gp_pad = jnp.pad(gene_pathway, ((0, 0), (0, P_PAD - NUM_PATH)))
    gid3 = gene_ids.reshape(NW, NCH_B, 128)
    cid2 = context_ids.reshape(NW, PER_TILE_C)
    xg_flat, pm_flat, ctx = _bgather(X_prop, gp_pad, treat_embed, gid3, cid2)
    xg = xg_flat.reshape(B, M, EMBED)
    pmask = pm_flat.reshape(B, M, P_PAD)

    return _dense_stage(xg, pmask, ctx, path_w1, path_b1, path_w2, path_b2,
                        latent_w, latent_b, risk_w, risk_b)
